# Initial kernel scaffold; baseline (speedup 1.0000x reference)
import jax
import jax.numpy as jnp
from jax.experimental import pallas as pl


def kernel(mq_x, sq_x, edge_index_mq2sq, edge_index_sq2mq, edge_label_index, W_lin, b_lin, Wl_m2s, bl_m2s, Wr_m2s, Wl_s2m, bl_s2m, Wr_s2m):
    raise NotImplementedError("write your pallas kernel here")


# speedup vs baseline: 2.5244x; 2.5244x over previous
import functools

import jax
import jax.numpy as jnp
from jax import lax
from jax.experimental import pallas as pl
from jax.experimental.pallas import tpu as pltpu
from jax.experimental.pallas import tpu_sc as plsc

_N = 10000
_E = 160000
_EL = 100000
_H = 256
_HH = 128
_DIN = 768
_K = 128
_NC = 2
_NS = 16
_ROWS_PER_TILE = 624
_ROWS_TAIL = _N - _ROWS_PER_TILE * _NS
_NCHUNK = _E // _K
_FULL_ITERS = _NCHUNK // _NS
_REM = _NCHUNK - _FULL_ITERS * _NS

@functools.lru_cache(maxsize=None)
def _mesh():
  return plsc.VectorSubcoreMesh(
      core_axis_name="c", subcore_axis_name="s",
      num_cores=_NC, num_subcores=_NS)


def _tile_rows(s, src, dst):
  r0 = s * _ROWS_PER_TILE
  pltpu.sync_copy(src.at[pl.ds(r0, _ROWS_PER_TILE)],
                  dst.at[pl.ds(r0, _ROWS_PER_TILE)])

  @pl.when(s == 0)
  def _():
    tail0 = _ROWS_PER_TILE * _NS
    pltpu.sync_copy(src.at[pl.ds(tail0, _ROWS_TAIL)],
                    dst.at[pl.ds(tail0, _ROWS_TAIL)])


def _edge_chunks(s, chunk):
  def iter_body(i, carry):
    chunk(s + i * _NS)
    return carry
  lax.fori_loop(0, _FULL_ITERS, iter_body, 0)

  @pl.when(s < _REM)
  def _():
    chunk(_FULL_ITERS * _NS + s)


@functools.lru_cache(maxsize=None)
def _make_segsum():
  def body(x_lo, x_hi, src, dst, zeros2d,
           out_lo, out_hi, src_v, dst_v, rows_v, acc_s, sem):
    c = lax.axis_index("c")
    s = lax.axis_index("s")

    _tile_rows(s, zeros2d, acc_s)
    plsc.subcore_barrier()

    def chunk(cid, x_h):
      base = cid * _K
      pltpu.sync_copy(src.at[pl.ds(base, _K)], src_v)
      pltpu.sync_copy(dst.at[pl.ds(base, _K)], dst_v)
      pltpu.async_copy(x_h.at[src_v], rows_v, sem).wait()
      pltpu.sync_copy(rows_v, acc_s.at[dst_v], add=True)

    @pl.when(c == 0)
    def _():
      _edge_chunks(s, lambda cid: chunk(cid, x_lo))

    @pl.when(c == 1)
    def _():
      _edge_chunks(s, lambda cid: chunk(cid, x_hi))

    plsc.subcore_barrier()

    @pl.when(c == 0)
    def _():
      _tile_rows(s, acc_s, out_lo)

    @pl.when(c == 1)
    def _():
      _tile_rows(s, acc_s, out_hi)

  return pl.kernel(
      body,
      out_type=[
          jax.ShapeDtypeStruct((_N, _HH), jnp.float32),
          jax.ShapeDtypeStruct((_N, _HH), jnp.float32),
      ],
      mesh=_mesh(),
      scratch_types=[
          pltpu.VMEM((_K,), jnp.int32),
          pltpu.VMEM((_K,), jnp.int32),
          pltpu.VMEM((_K, _HH), jnp.float32),
          pltpu.VMEM_SHARED((_N, _HH), jnp.float32),
          pltpu.SemaphoreType.DMA,
      ],
      name="segsum")


@functools.lru_cache(maxsize=None)
def _make_counts():
  def body(dst_a, dst_b, zeros2d, ones128,
           out_a, out_b, dst_v, rows_v, acc_s):
    c = lax.axis_index("c")
    s = lax.axis_index("s")

    _tile_rows(s, zeros2d, acc_s)
    pltpu.sync_copy(ones128, rows_v)
    plsc.subcore_barrier()

    def chunk(cid, dst_e):
      pltpu.sync_copy(dst_e.at[pl.ds(cid * _K, _K)], dst_v)
      pltpu.sync_copy(rows_v, acc_s.at[dst_v], add=True)

    @pl.when(c == 0)
    def _():
      _edge_chunks(s, lambda cid: chunk(cid, dst_a))

    @pl.when(c == 1)
    def _():
      _edge_chunks(s, lambda cid: chunk(cid, dst_b))

    plsc.subcore_barrier()

    @pl.when(c == 0)
    def _():
      _tile_rows(s, acc_s, out_a)

    @pl.when(c == 1)
    def _():
      _tile_rows(s, acc_s, out_b)

  return pl.kernel(
      body,
      out_type=[
          jax.ShapeDtypeStruct((_N, _HH), jnp.float32),
          jax.ShapeDtypeStruct((_N, _HH), jnp.float32),
      ],
      mesh=_mesh(),
      scratch_types=[
          pltpu.VMEM((_K,), jnp.int32),
          pltpu.VMEM((_K, _HH), jnp.float32),
          pltpu.VMEM_SHARED((_N, _HH), jnp.float32),
      ],
      name="edge_counts")

_HCHUNKS = _EL // _K
_HTAIL = _EL - _HCHUNKS * _K
_HITERS = -(-_HCHUNKS // (_NC * _NS))


def _head_body(mq_lo, mq_hi, sq_lo, sq_hi, eu, em, out,
               ui, mi, ulo, uhi, mlo, mhi, res, sem):
  w = lax.axis_index("s") * _NC + lax.axis_index("c")
  lanes = lax.iota(jnp.int32, 16)

  def group_dot(g):
    rows = lanes + g * 16

    def j_body(j, acc):
      for u in range(8):
        col = jnp.full((16,), j * 8 + u, jnp.int32)
        acc = acc + (plsc.load_gather(ulo, [rows, col])
                     * plsc.load_gather(mlo, [rows, col]))
        acc = acc + (plsc.load_gather(uhi, [rows, col])
                     * plsc.load_gather(mhi, [rows, col]))
      return acc

    acc = lax.fori_loop(0, _HH // 8, j_body, jnp.zeros((16,), jnp.float32))
    res[pl.ds(g * 16, 16)] = acc

  def do_chunk(base, n):
    pltpu.sync_copy(eu.at[pl.ds(base, n)], ui.at[pl.ds(0, n)])
    pltpu.sync_copy(em.at[pl.ds(base, n)], mi.at[pl.ds(0, n)])
    pltpu.async_copy(mq_lo.at[ui], ulo, sem).wait()
    pltpu.async_copy(mq_hi.at[ui], uhi, sem).wait()
    pltpu.async_copy(sq_lo.at[mi], mlo, sem).wait()
    pltpu.async_copy(sq_hi.at[mi], mhi, sem).wait()

    def g_body(g, carry):
      group_dot(g)
      return carry
    lax.fori_loop(0, n // 16, g_body, 0)
    pltpu.sync_copy(res.at[pl.ds(0, n)], out.at[pl.ds(base, n)])

  def iter_body(i, carry):
    cid = w + i * (_NC * _NS)

    @pl.when(cid < _HCHUNKS)
    def _():
      do_chunk(cid * _K, _K)
    return carry

  lax.fori_loop(0, _HITERS, iter_body, 0)

  @pl.when(w == 0)
  def _():
    do_chunk(_HCHUNKS * _K, _HTAIL)


@functools.lru_cache(maxsize=None)
def _make_head():
  return pl.kernel(
    _head_body,
    out_type=jax.ShapeDtypeStruct((_EL,), jnp.float32),
    mesh=_mesh(),
    scratch_types=[
        pltpu.VMEM((_K,), jnp.int32),
        pltpu.VMEM((_K,), jnp.int32),
        pltpu.VMEM((_K, _HH), jnp.float32),
        pltpu.VMEM((_K, _HH), jnp.float32),
        pltpu.VMEM((_K, _HH), jnp.float32),
        pltpu.VMEM((_K, _HH), jnp.float32),
        pltpu.VMEM((_K,), jnp.float32),
        pltpu.SemaphoreType.DMA,
    ],
    compiler_params=pltpu.CompilerParams(needs_layout_passes=False),
    name="edge_dot_head")


_BN = 2000


def _linear_body(x_ref, w_ref, b_ref, olo_ref, ohi_ref):
  y = (jnp.dot(x_ref[...], w_ref[...], preferred_element_type=jnp.float32)
       + b_ref[...])
  olo_ref[...] = y[:, :_HH]
  ohi_ref[...] = y[:, _HH:]


def _linear(x, W, b):
  return pl.pallas_call(
      _linear_body,
      grid=(_N // _BN,),
      in_specs=[
          pl.BlockSpec((_BN, _DIN), lambda i: (i, 0)),
          pl.BlockSpec((_DIN, _H), lambda i: (0, 0)),
          pl.BlockSpec((1, _H), lambda i: (0, 0)),
      ],
      out_specs=[
          pl.BlockSpec((_BN, _HH), lambda i: (i, 0)),
          pl.BlockSpec((_BN, _HH), lambda i: (i, 0)),
      ],
      out_shape=[
          jax.ShapeDtypeStruct((_N, _HH), jnp.float32),
          jax.ShapeDtypeStruct((_N, _HH), jnp.float32),
      ],
  )(x, W, b.reshape(1, _H))


def _make_update(relu: bool):
  def body(agg_lo_ref, agg_hi_ref, cnt_ref, x_lo_ref, x_hi_ref,
           wl_lo_ref, wl_hi_ref, wr_lo_ref, wr_hi_ref, b_ref,
           olo_ref, ohi_ref):
    inv = 1.0 / jnp.maximum(cnt_ref[...], 1.0)
    y = jnp.dot(agg_lo_ref[...] * inv, wl_lo_ref[...],
                preferred_element_type=jnp.float32)
    y += jnp.dot(agg_hi_ref[...] * inv, wl_hi_ref[...],
                 preferred_element_type=jnp.float32)
    y += jnp.dot(x_lo_ref[...], wr_lo_ref[...],
                 preferred_element_type=jnp.float32)
    y += jnp.dot(x_hi_ref[...], wr_hi_ref[...],
                 preferred_element_type=jnp.float32)
    y += b_ref[...]
    if relu:
      y = jnp.maximum(y, 0.0)
    olo_ref[...] = y[:, :_HH]
    ohi_ref[...] = y[:, _HH:]

  def update(agg_lo, agg_hi, cntb, x_lo, x_hi, Wl, bl, Wr):
    rows = lambda i: (i, 0)
    bcast = lambda i: (0, 0)
    return pl.pallas_call(
        body,
        grid=(_N // _BN,),
        in_specs=[
            pl.BlockSpec((_BN, _HH), rows),
            pl.BlockSpec((_BN, _HH), rows),
            pl.BlockSpec((_BN, _HH), rows),
            pl.BlockSpec((_BN, _HH), rows),
            pl.BlockSpec((_BN, _HH), rows),
            pl.BlockSpec((_HH, _H), bcast),
            pl.BlockSpec((_HH, _H), bcast),
            pl.BlockSpec((_HH, _H), bcast),
            pl.BlockSpec((_HH, _H), bcast),
            pl.BlockSpec((1, _H), bcast),
        ],
        out_specs=[
            pl.BlockSpec((_BN, _HH), rows),
            pl.BlockSpec((_BN, _HH), rows),
        ],
        out_shape=[
            jax.ShapeDtypeStruct((_N, _HH), jnp.float32),
            jax.ShapeDtypeStruct((_N, _HH), jnp.float32),
        ],
        name="update_relu" if relu else "update",
    )(agg_lo, agg_hi, cntb, x_lo, x_hi,
      Wl[:_HH], Wl[_HH:], Wr[:_HH], Wr[_HH:], bl.reshape(1, _H))

  return update


_update_relu = _make_update(True)
_update_plain = _make_update(False)


def kernel(mq_x, sq_x, edge_index_mq2sq, edge_index_sq2mq, edge_label_index,
           W_lin, b_lin, Wl_m2s, bl_m2s, Wr_m2s, Wl_s2m, bl_s2m, Wr_s2m):
  src_m2s, dst_m2s = edge_index_mq2sq[0], edge_index_mq2sq[1]
  src_s2m, dst_s2m = edge_index_sq2mq[0], edge_index_sq2mq[1]
  eu, em = edge_label_index[0], edge_label_index[1]

  zeros2d = jnp.zeros((_N, _HH), jnp.float32)
  ones128 = jnp.ones((_K, _HH), jnp.float32)

  mq_lo, mq_hi = _linear(mq_x, W_lin, b_lin)
  sq_lo, sq_hi = _linear(sq_x, W_lin, b_lin)

  cntb_m2s, cntb_s2m = _make_counts()(dst_m2s, dst_s2m, zeros2d, ones128)

  for l in range(4):
    agg_s_lo, agg_s_hi = _make_segsum()(
        mq_lo, mq_hi, src_m2s, dst_m2s, zeros2d)
    agg_m_lo, agg_m_hi = _make_segsum()(
        sq_lo, sq_hi, src_s2m, dst_s2m, zeros2d)
    upd = _update_relu if l == 0 else _update_plain
    sq_lo_n, sq_hi_n = upd(agg_s_lo, agg_s_hi, cntb_m2s, sq_lo, sq_hi,
                           Wl_m2s[l], bl_m2s[l], Wr_m2s[l])
    mq_lo_n, mq_hi_n = upd(agg_m_lo, agg_m_hi, cntb_s2m, mq_lo, mq_hi,
                           Wl_s2m[l], bl_s2m[l], Wr_s2m[l])
    mq_lo, mq_hi, sq_lo, sq_hi = mq_lo_n, mq_hi_n, sq_lo_n, sq_hi_n

  return _make_head()(mq_lo, mq_hi, sq_lo, sq_hi, eu, em)


# speedup vs baseline: 3.3301x; 1.3192x over previous
import functools

import jax
import jax.numpy as jnp
from jax import lax
from jax.experimental import pallas as pl
from jax.experimental.pallas import tpu as pltpu
from jax.experimental.pallas import tpu_sc as plsc

_N = 10000
_E = 160000
_EL = 100000
_H = 256
_HH = 128
_DIN = 768
_K = 128
_NC = 2
_NS = 16
_ROWS_PER_TILE = 624
_ROWS_TAIL = _N - _ROWS_PER_TILE * _NS
_NCHUNK = _E // _K
_FULL_ITERS = _NCHUNK // _NS
_REM = _NCHUNK - _FULL_ITERS * _NS

@functools.lru_cache(maxsize=None)
def _mesh():
  return plsc.VectorSubcoreMesh(
      core_axis_name="c", subcore_axis_name="s",
      num_cores=_NC, num_subcores=_NS)


def _tile_rows(s, src, dst):
  r0 = s * _ROWS_PER_TILE
  pltpu.sync_copy(src.at[pl.ds(r0, _ROWS_PER_TILE)],
                  dst.at[pl.ds(r0, _ROWS_PER_TILE)])

  @pl.when(s == 0)
  def _():
    tail0 = _ROWS_PER_TILE * _NS
    pltpu.sync_copy(src.at[pl.ds(tail0, _ROWS_TAIL)],
                    dst.at[pl.ds(tail0, _ROWS_TAIL)])


def _edge_chunks(s, chunk):
  def iter_body(i, carry):
    chunk(s + i * _NS)
    return carry
  lax.fori_loop(0, _FULL_ITERS, iter_body, 0)

  @pl.when(s < _REM)
  def _():
    chunk(_FULL_ITERS * _NS + s)


@functools.lru_cache(maxsize=None)
def _make_segsum():
  def body(x_lo, x_hi, src, dst, zeros2d,
           out_lo, out_hi, src_v, dst_v, rows_v, acc_s, sem):
    c = lax.axis_index("c")
    s = lax.axis_index("s")

    _tile_rows(s, zeros2d, acc_s)
    plsc.subcore_barrier()

    def chunk(cid, x_h):
      base = cid * _K
      pltpu.sync_copy(src.at[pl.ds(base, _K)], src_v)
      pltpu.sync_copy(dst.at[pl.ds(base, _K)], dst_v)
      pltpu.async_copy(x_h.at[src_v], rows_v, sem).wait()
      pltpu.sync_copy(rows_v, acc_s.at[dst_v], add=True)

    @pl.when(c == 0)
    def _():
      _edge_chunks(s, lambda cid: chunk(cid, x_lo))

    @pl.when(c == 1)
    def _():
      _edge_chunks(s, lambda cid: chunk(cid, x_hi))

    plsc.subcore_barrier()

    @pl.when(c == 0)
    def _():
      _tile_rows(s, acc_s, out_lo)

    @pl.when(c == 1)
    def _():
      _tile_rows(s, acc_s, out_hi)

  return pl.kernel(
      body,
      out_type=[
          jax.ShapeDtypeStruct((_N, _HH), jnp.float32),
          jax.ShapeDtypeStruct((_N, _HH), jnp.float32),
      ],
      mesh=_mesh(),
      scratch_types=[
          pltpu.VMEM((_K,), jnp.int32),
          pltpu.VMEM((_K,), jnp.int32),
          pltpu.VMEM((_K, _HH), jnp.float32),
          pltpu.VMEM_SHARED((_N, _HH), jnp.float32),
          pltpu.SemaphoreType.DMA,
      ],
      name="segsum")


@functools.lru_cache(maxsize=None)
def _make_counts():
  def body(dst_a, dst_b, zeros2d, ones128,
           out_a, out_b, dst_v, rows_v, acc_s):
    c = lax.axis_index("c")
    s = lax.axis_index("s")

    _tile_rows(s, zeros2d, acc_s)
    pltpu.sync_copy(ones128, rows_v)
    plsc.subcore_barrier()

    def chunk(cid, dst_e):
      pltpu.sync_copy(dst_e.at[pl.ds(cid * _K, _K)], dst_v)
      pltpu.sync_copy(rows_v, acc_s.at[dst_v], add=True)

    @pl.when(c == 0)
    def _():
      _edge_chunks(s, lambda cid: chunk(cid, dst_a))

    @pl.when(c == 1)
    def _():
      _edge_chunks(s, lambda cid: chunk(cid, dst_b))

    plsc.subcore_barrier()

    @pl.when(c == 0)
    def _():
      _tile_rows(s, acc_s, out_a)

    @pl.when(c == 1)
    def _():
      _tile_rows(s, acc_s, out_b)

  return pl.kernel(
      body,
      out_type=[
          jax.ShapeDtypeStruct((_N, _HH), jnp.float32),
          jax.ShapeDtypeStruct((_N, _HH), jnp.float32),
      ],
      mesh=_mesh(),
      scratch_types=[
          pltpu.VMEM((_K,), jnp.int32),
          pltpu.VMEM((_K, _HH), jnp.float32),
          pltpu.VMEM_SHARED((_N, _HH), jnp.float32),
      ],
      name="edge_counts")

_HCHUNKS = _EL // _K
_HTAIL = _EL - _HCHUNKS * _K
_HITERS = -(-_HCHUNKS // (_NC * _NS))


def _head_body(mq_lo, mq_hi, sq_lo, sq_hi, eu, em, pout,
               ui, mi, ulo, uhi, mlo, mhi, res, sem):
  w = lax.axis_index("s") * _NC + lax.axis_index("c")

  def edge_partial(r):
    acc = ulo[r, pl.ds(0, 16)] * mlo[r, pl.ds(0, 16)]
    for j in range(1, _HH // 16):
      sl = pl.ds(j * 16, 16)
      acc = acc + ulo[r, sl] * mlo[r, sl]
    for j in range(_HH // 16):
      sl = pl.ds(j * 16, 16)
      acc = acc + uhi[r, sl] * mhi[r, sl]
    res[r // 8, pl.ds((r % 8) * 16, 16)] = acc

  def do_chunk(base, n):
    pltpu.sync_copy(eu.at[pl.ds(base, n)], ui.at[pl.ds(0, n)])
    pltpu.sync_copy(em.at[pl.ds(base, n)], mi.at[pl.ds(0, n)])
    pltpu.async_copy(mq_lo.at[ui], ulo, sem).wait()
    pltpu.async_copy(mq_hi.at[ui], uhi, sem).wait()
    pltpu.async_copy(sq_lo.at[mi], mlo, sem).wait()
    pltpu.async_copy(sq_hi.at[mi], mhi, sem).wait()

    def r_body(r, carry):
      edge_partial(r)
      return carry
    lax.fori_loop(0, n, r_body, 0)
    pltpu.sync_copy(res.at[pl.ds(0, n // 8)],
                    pout.at[pl.ds(pl.multiple_of(base // 8, 8), n // 8)])

  def iter_body(i, carry):
    cid = w + i * (_NC * _NS)

    @pl.when(cid < _HCHUNKS)
    def _():
      do_chunk(cid * _K, _K)
    return carry

  lax.fori_loop(0, _HITERS, iter_body, 0)

  @pl.when(w == 0)
  def _():
    do_chunk(_HCHUNKS * _K, _HTAIL)


@functools.lru_cache(maxsize=None)
def _make_head():
  return pl.kernel(
    _head_body,
    out_type=jax.ShapeDtypeStruct((_EL // 8, _HH), jnp.float32),
    mesh=_mesh(),
    scratch_types=[
        pltpu.VMEM((_K,), jnp.int32),
        pltpu.VMEM((_K,), jnp.int32),
        pltpu.VMEM((_K, _HH), jnp.float32),
        pltpu.VMEM((_K, _HH), jnp.float32),
        pltpu.VMEM((_K, _HH), jnp.float32),
        pltpu.VMEM((_K, _HH), jnp.float32),
        pltpu.VMEM((_K // 8, _HH), jnp.float32),
        pltpu.SemaphoreType.DMA,
    ],
    compiler_params=pltpu.CompilerParams(needs_layout_passes=False),
    name="edge_dot_head")


def _headsum_body(p_ref, s_ref, o_ref):
  o_ref[...] = jnp.dot(p_ref[...], s_ref[...],
                       preferred_element_type=jnp.float32)


def _head_reduce(pout, sel):
  return pl.pallas_call(
      _headsum_body,
      out_shape=jax.ShapeDtypeStruct((_EL // 8, 8), jnp.float32),
      name="head_reduce",
  )(pout, sel)


_BN = 2000


def _linear_body(x_ref, w_ref, b_ref, olo_ref, ohi_ref):
  y = (jnp.dot(x_ref[...], w_ref[...], preferred_element_type=jnp.float32)
       + b_ref[...])
  olo_ref[...] = y[:, :_HH]
  ohi_ref[...] = y[:, _HH:]


def _linear(x, W, b):
  return pl.pallas_call(
      _linear_body,
      grid=(_N // _BN,),
      in_specs=[
          pl.BlockSpec((_BN, _DIN), lambda i: (i, 0)),
          pl.BlockSpec((_DIN, _H), lambda i: (0, 0)),
          pl.BlockSpec((1, _H), lambda i: (0, 0)),
      ],
      out_specs=[
          pl.BlockSpec((_BN, _HH), lambda i: (i, 0)),
          pl.BlockSpec((_BN, _HH), lambda i: (i, 0)),
      ],
      out_shape=[
          jax.ShapeDtypeStruct((_N, _HH), jnp.float32),
          jax.ShapeDtypeStruct((_N, _HH), jnp.float32),
      ],
  )(x, W, b.reshape(1, _H))


def _make_update(relu: bool):
  def body(agg_lo_ref, agg_hi_ref, cnt_ref, x_lo_ref, x_hi_ref,
           wl_lo_ref, wl_hi_ref, wr_lo_ref, wr_hi_ref, b_ref,
           olo_ref, ohi_ref):
    inv = 1.0 / jnp.maximum(cnt_ref[...], 1.0)
    y = jnp.dot(agg_lo_ref[...] * inv, wl_lo_ref[...],
                preferred_element_type=jnp.float32)
    y += jnp.dot(agg_hi_ref[...] * inv, wl_hi_ref[...],
                 preferred_element_type=jnp.float32)
    y += jnp.dot(x_lo_ref[...], wr_lo_ref[...],
                 preferred_element_type=jnp.float32)
    y += jnp.dot(x_hi_ref[...], wr_hi_ref[...],
                 preferred_element_type=jnp.float32)
    y += b_ref[...]
    if relu:
      y = jnp.maximum(y, 0.0)
    olo_ref[...] = y[:, :_HH]
    ohi_ref[...] = y[:, _HH:]

  def update(agg_lo, agg_hi, cntb, x_lo, x_hi, Wl, bl, Wr):
    rows = lambda i: (i, 0)
    bcast = lambda i: (0, 0)
    return pl.pallas_call(
        body,
        grid=(_N // _BN,),
        in_specs=[
            pl.BlockSpec((_BN, _HH), rows),
            pl.BlockSpec((_BN, _HH), rows),
            pl.BlockSpec((_BN, _HH), rows),
            pl.BlockSpec((_BN, _HH), rows),
            pl.BlockSpec((_BN, _HH), rows),
            pl.BlockSpec((_HH, _H), bcast),
            pl.BlockSpec((_HH, _H), bcast),
            pl.BlockSpec((_HH, _H), bcast),
            pl.BlockSpec((_HH, _H), bcast),
            pl.BlockSpec((1, _H), bcast),
        ],
        out_specs=[
            pl.BlockSpec((_BN, _HH), rows),
            pl.BlockSpec((_BN, _HH), rows),
        ],
        out_shape=[
            jax.ShapeDtypeStruct((_N, _HH), jnp.float32),
            jax.ShapeDtypeStruct((_N, _HH), jnp.float32),
        ],
        name="update_relu" if relu else "update",
    )(agg_lo, agg_hi, cntb, x_lo, x_hi,
      Wl[:_HH], Wl[_HH:], Wr[:_HH], Wr[_HH:], bl.reshape(1, _H))

  return update


_update_relu = _make_update(True)
_update_plain = _make_update(False)


def kernel(mq_x, sq_x, edge_index_mq2sq, edge_index_sq2mq, edge_label_index,
           W_lin, b_lin, Wl_m2s, bl_m2s, Wr_m2s, Wl_s2m, bl_s2m, Wr_s2m):
  src_m2s, dst_m2s = edge_index_mq2sq[0], edge_index_mq2sq[1]
  src_s2m, dst_s2m = edge_index_sq2mq[0], edge_index_sq2mq[1]
  eu, em = edge_label_index[0], edge_label_index[1]

  zeros2d = jnp.zeros((_N, _HH), jnp.float32)
  ones128 = jnp.ones((_K, _HH), jnp.float32)

  mq_lo, mq_hi = _linear(mq_x, W_lin, b_lin)
  sq_lo, sq_hi = _linear(sq_x, W_lin, b_lin)

  cntb_m2s, cntb_s2m = _make_counts()(dst_m2s, dst_s2m, zeros2d, ones128)

  for l in range(4):
    agg_s_lo, agg_s_hi = _make_segsum()(
        mq_lo, mq_hi, src_m2s, dst_m2s, zeros2d)
    agg_m_lo, agg_m_hi = _make_segsum()(
        sq_lo, sq_hi, src_s2m, dst_s2m, zeros2d)
    upd = _update_relu if l == 0 else _update_plain
    sq_lo_n, sq_hi_n = upd(agg_s_lo, agg_s_hi, cntb_m2s, sq_lo, sq_hi,
                           Wl_m2s[l], bl_m2s[l], Wr_m2s[l])
    mq_lo_n, mq_hi_n = upd(agg_m_lo, agg_m_hi, cntb_s2m, mq_lo, mq_hi,
                           Wl_s2m[l], bl_s2m[l], Wr_s2m[l])
    mq_lo, mq_hi, sq_lo, sq_hi = mq_lo_n, mq_hi_n, sq_lo_n, sq_hi_n

  pout = _make_head()(mq_lo, mq_hi, sq_lo, sq_hi, eu, em)
  sel = jnp.repeat(jnp.eye(8, dtype=jnp.float32), 16, axis=0)
  return _head_reduce(pout, sel).reshape(_EL)


# speedup vs baseline: 4.7775x; 1.4347x over previous
import functools

import jax
import jax.numpy as jnp
from jax import lax
from jax.experimental import pallas as pl
from jax.experimental.pallas import tpu as pltpu
from jax.experimental.pallas import tpu_sc as plsc

_N = 10000
_E = 160000
_EL = 100000
_H = 256
_HH = 128
_DIN = 768
_K = 128
_NC = 2
_NS = 16
_ROWS_PER_TILE = 624
_ROWS_TAIL = _N - _ROWS_PER_TILE * _NS
_NCHUNK = _E // _K
_FULL_ITERS = _NCHUNK // _NS
_REM = _NCHUNK - _FULL_ITERS * _NS

@functools.lru_cache(maxsize=None)
def _mesh():
  return plsc.VectorSubcoreMesh(
      core_axis_name="c", subcore_axis_name="s",
      num_cores=_NC, num_subcores=_NS)


def _tile_rows(s, src, dst):
  r0 = s * _ROWS_PER_TILE
  pltpu.sync_copy(src.at[pl.ds(r0, _ROWS_PER_TILE)],
                  dst.at[pl.ds(r0, _ROWS_PER_TILE)])

  @pl.when(s == 0)
  def _():
    tail0 = _ROWS_PER_TILE * _NS
    pltpu.sync_copy(src.at[pl.ds(tail0, _ROWS_TAIL)],
                    dst.at[pl.ds(tail0, _ROWS_TAIL)])


def _edge_chunks(s, chunk):
  def iter_body(i, carry):
    chunk(s + i * _NS)
    return carry
  lax.fori_loop(0, _FULL_ITERS, iter_body, 0)

  @pl.when(s < _REM)
  def _():
    chunk(_FULL_ITERS * _NS + s)


@functools.lru_cache(maxsize=None)
def _make_segsum():
  def body(x_lo, x_hi, src, dst, zeros2d, out_lo, out_hi,
           sv0, dv0, rv0, sv1, dv1, rv1, acc_s, g0, g1, s0, s1):
    c = lax.axis_index("c")
    s = lax.axis_index("s")

    _tile_rows(s, zeros2d, acc_s)
    plsc.subcore_barrier()

    def work(x_h):
      def lg(i, sv, dv, rv, gsem):
        base = (s + i * _NS) * _K
        pltpu.sync_copy(src.at[pl.ds(base, _K)], sv)
        pltpu.sync_copy(dst.at[pl.ds(base, _K)], dv)
        pltpu.async_copy(x_h.at[sv], rv, gsem)

      def wait64k(rv, sem):
        pltpu.make_async_copy(x_h.at[pl.ds(0, _K)], rv, sem).wait()

      def sc(rv, dv, ssem):
        pltpu.async_copy(rv, acc_s.at[dv], ssem, add=True)

      lg(0, sv0, dv0, rv0, g0)
      lg(1, sv1, dv1, rv1, g1)
      wait64k(rv0, g0)
      sc(rv0, dv0, s0)
      wait64k(rv1, g1)
      sc(rv1, dv1, s1)

      def pair(p, carry):
        wait64k(rv0, s0)
        lg(2 * p, sv0, dv0, rv0, g0)
        wait64k(rv1, s1)
        lg(2 * p + 1, sv1, dv1, rv1, g1)
        wait64k(rv0, g0)
        sc(rv0, dv0, s0)
        wait64k(rv1, g1)
        sc(rv1, dv1, s1)
        return carry
      lax.fori_loop(1, _FULL_ITERS // 2, pair, 0)

      @pl.when(s < _REM)
      def _():
        wait64k(rv0, s0)
        base = (_FULL_ITERS * _NS + s) * _K
        pltpu.sync_copy(src.at[pl.ds(base, _K)], sv0)
        pltpu.sync_copy(dst.at[pl.ds(base, _K)], dv0)
        pltpu.async_copy(x_h.at[sv0], rv0, g0)
        wait64k(rv0, g0)
        sc(rv0, dv0, s0)

      wait64k(rv0, s0)
      wait64k(rv1, s1)

    @pl.when(c == 0)
    def _():
      work(x_lo)

    @pl.when(c == 1)
    def _():
      work(x_hi)

    plsc.subcore_barrier()

    @pl.when(c == 0)
    def _():
      _tile_rows(s, acc_s, out_lo)

    @pl.when(c == 1)
    def _():
      _tile_rows(s, acc_s, out_hi)

  return pl.kernel(
      body,
      out_type=[
          jax.ShapeDtypeStruct((_N, _HH), jnp.float32),
          jax.ShapeDtypeStruct((_N, _HH), jnp.float32),
      ],
      mesh=_mesh(),
      scratch_types=[
          pltpu.VMEM((_K,), jnp.int32),
          pltpu.VMEM((_K,), jnp.int32),
          pltpu.VMEM((_K, _HH), jnp.float32),
          pltpu.VMEM((_K,), jnp.int32),
          pltpu.VMEM((_K,), jnp.int32),
          pltpu.VMEM((_K, _HH), jnp.float32),
          pltpu.VMEM_SHARED((_N, _HH), jnp.float32),
          pltpu.SemaphoreType.DMA,
          pltpu.SemaphoreType.DMA,
          pltpu.SemaphoreType.DMA,
          pltpu.SemaphoreType.DMA,
      ],
      name="segsum")


@functools.lru_cache(maxsize=None)
def _make_counts():
  def body(dst_a, dst_b, zeros2d, ones128,
           out_a, out_b, dst_v, rows_v, acc_s):
    c = lax.axis_index("c")
    s = lax.axis_index("s")

    _tile_rows(s, zeros2d, acc_s)
    pltpu.sync_copy(ones128, rows_v)
    plsc.subcore_barrier()

    def chunk(cid, dst_e):
      pltpu.sync_copy(dst_e.at[pl.ds(cid * _K, _K)], dst_v)
      pltpu.sync_copy(rows_v, acc_s.at[dst_v], add=True)

    @pl.when(c == 0)
    def _():
      _edge_chunks(s, lambda cid: chunk(cid, dst_a))

    @pl.when(c == 1)
    def _():
      _edge_chunks(s, lambda cid: chunk(cid, dst_b))

    plsc.subcore_barrier()

    @pl.when(c == 0)
    def _():
      _tile_rows(s, acc_s, out_a)

    @pl.when(c == 1)
    def _():
      _tile_rows(s, acc_s, out_b)

  return pl.kernel(
      body,
      out_type=[
          jax.ShapeDtypeStruct((_N, _HH), jnp.float32),
          jax.ShapeDtypeStruct((_N, _HH), jnp.float32),
      ],
      mesh=_mesh(),
      scratch_types=[
          pltpu.VMEM((_K,), jnp.int32),
          pltpu.VMEM((_K, _HH), jnp.float32),
          pltpu.VMEM_SHARED((_N, _HH), jnp.float32),
      ],
      name="edge_counts")

_HCHUNKS = _EL // _K
_HTAIL = _EL - _HCHUNKS * _K
_HITERS = -(-_HCHUNKS // (_NC * _NS))


def _head_body(mq_lo, mq_hi, sq_lo, sq_hi, eu, em, pout,
               ui, mi, ulo, uhi, mlo, mhi, res, sem):
  w = lax.axis_index("s") * _NC + lax.axis_index("c")

  def edge_partial(r):
    acc = ulo[r, pl.ds(0, 16)] * mlo[r, pl.ds(0, 16)]
    for j in range(1, _HH // 16):
      sl = pl.ds(j * 16, 16)
      acc = acc + ulo[r, sl] * mlo[r, sl]
    for j in range(_HH // 16):
      sl = pl.ds(j * 16, 16)
      acc = acc + uhi[r, sl] * mhi[r, sl]
    res[r // 8, pl.ds((r % 8) * 16, 16)] = acc

  def do_chunk(base, n):
    pltpu.sync_copy(eu.at[pl.ds(base, n)], ui.at[pl.ds(0, n)])
    pltpu.sync_copy(em.at[pl.ds(base, n)], mi.at[pl.ds(0, n)])
    pltpu.async_copy(mq_lo.at[ui], ulo, sem).wait()
    pltpu.async_copy(mq_hi.at[ui], uhi, sem).wait()
    pltpu.async_copy(sq_lo.at[mi], mlo, sem).wait()
    pltpu.async_copy(sq_hi.at[mi], mhi, sem).wait()

    def r_body(r, carry):
      edge_partial(r)
      return carry
    lax.fori_loop(0, n, r_body, 0)
    pltpu.sync_copy(res.at[pl.ds(0, n // 8)],
                    pout.at[pl.ds(pl.multiple_of(base // 8, 8), n // 8)])

  def iter_body(i, carry):
    cid = w + i * (_NC * _NS)

    @pl.when(cid < _HCHUNKS)
    def _():
      do_chunk(cid * _K, _K)
    return carry

  lax.fori_loop(0, _HITERS, iter_body, 0)

  @pl.when(w == 0)
  def _():
    do_chunk(_HCHUNKS * _K, _HTAIL)


@functools.lru_cache(maxsize=None)
def _make_head():
  return pl.kernel(
    _head_body,
    out_type=jax.ShapeDtypeStruct((_EL // 8, _HH), jnp.float32),
    mesh=_mesh(),
    scratch_types=[
        pltpu.VMEM((_K,), jnp.int32),
        pltpu.VMEM((_K,), jnp.int32),
        pltpu.VMEM((_K, _HH), jnp.float32),
        pltpu.VMEM((_K, _HH), jnp.float32),
        pltpu.VMEM((_K, _HH), jnp.float32),
        pltpu.VMEM((_K, _HH), jnp.float32),
        pltpu.VMEM((_K // 8, _HH), jnp.float32),
        pltpu.SemaphoreType.DMA,
    ],
    compiler_params=pltpu.CompilerParams(needs_layout_passes=False),
    name="edge_dot_head")


def _headsum_body(p_ref, s_ref, o_ref):
  o_ref[...] = jnp.dot(p_ref[...], s_ref[...],
                       preferred_element_type=jnp.float32)


def _head_reduce(pout, sel):
  return pl.pallas_call(
      _headsum_body,
      out_shape=jax.ShapeDtypeStruct((_EL // 8, 8), jnp.float32),
      name="head_reduce",
  )(pout, sel)


_BN = 2000


def _linear_body(x_ref, w_ref, b_ref, olo_ref, ohi_ref):
  y = (jnp.dot(x_ref[...], w_ref[...], preferred_element_type=jnp.float32)
       + b_ref[...])
  olo_ref[...] = y[:, :_HH]
  ohi_ref[...] = y[:, _HH:]


def _linear(x, W, b):
  return pl.pallas_call(
      _linear_body,
      grid=(_N // _BN,),
      in_specs=[
          pl.BlockSpec((_BN, _DIN), lambda i: (i, 0)),
          pl.BlockSpec((_DIN, _H), lambda i: (0, 0)),
          pl.BlockSpec((1, _H), lambda i: (0, 0)),
      ],
      out_specs=[
          pl.BlockSpec((_BN, _HH), lambda i: (i, 0)),
          pl.BlockSpec((_BN, _HH), lambda i: (i, 0)),
      ],
      out_shape=[
          jax.ShapeDtypeStruct((_N, _HH), jnp.float32),
          jax.ShapeDtypeStruct((_N, _HH), jnp.float32),
      ],
  )(x, W, b.reshape(1, _H))


def _make_update(relu: bool):
  def body(agg_lo_ref, agg_hi_ref, cnt_ref, x_lo_ref, x_hi_ref,
           wl_lo_ref, wl_hi_ref, wr_lo_ref, wr_hi_ref, b_ref,
           olo_ref, ohi_ref):
    inv = 1.0 / jnp.maximum(cnt_ref[...], 1.0)
    y = jnp.dot(agg_lo_ref[...] * inv, wl_lo_ref[...],
                preferred_element_type=jnp.float32)
    y += jnp.dot(agg_hi_ref[...] * inv, wl_hi_ref[...],
                 preferred_element_type=jnp.float32)
    y += jnp.dot(x_lo_ref[...], wr_lo_ref[...],
                 preferred_element_type=jnp.float32)
    y += jnp.dot(x_hi_ref[...], wr_hi_ref[...],
                 preferred_element_type=jnp.float32)
    y += b_ref[...]
    if relu:
      y = jnp.maximum(y, 0.0)
    olo_ref[...] = y[:, :_HH]
    ohi_ref[...] = y[:, _HH:]

  def update(agg_lo, agg_hi, cntb, x_lo, x_hi, Wl, bl, Wr):
    rows = lambda i: (i, 0)
    bcast = lambda i: (0, 0)
    return pl.pallas_call(
        body,
        grid=(_N // _BN,),
        in_specs=[
            pl.BlockSpec((_BN, _HH), rows),
            pl.BlockSpec((_BN, _HH), rows),
            pl.BlockSpec((_BN, _HH), rows),
            pl.BlockSpec((_BN, _HH), rows),
            pl.BlockSpec((_BN, _HH), rows),
            pl.BlockSpec((_HH, _H), bcast),
            pl.BlockSpec((_HH, _H), bcast),
            pl.BlockSpec((_HH, _H), bcast),
            pl.BlockSpec((_HH, _H), bcast),
            pl.BlockSpec((1, _H), bcast),
        ],
        out_specs=[
            pl.BlockSpec((_BN, _HH), rows),
            pl.BlockSpec((_BN, _HH), rows),
        ],
        out_shape=[
            jax.ShapeDtypeStruct((_N, _HH), jnp.float32),
            jax.ShapeDtypeStruct((_N, _HH), jnp.float32),
        ],
        name="update_relu" if relu else "update",
    )(agg_lo, agg_hi, cntb, x_lo, x_hi,
      Wl[:_HH], Wl[_HH:], Wr[:_HH], Wr[_HH:], bl.reshape(1, _H))

  return update


_update_relu = _make_update(True)
_update_plain = _make_update(False)


def kernel(mq_x, sq_x, edge_index_mq2sq, edge_index_sq2mq, edge_label_index,
           W_lin, b_lin, Wl_m2s, bl_m2s, Wr_m2s, Wl_s2m, bl_s2m, Wr_s2m):
  src_m2s, dst_m2s = edge_index_mq2sq[0], edge_index_mq2sq[1]
  src_s2m, dst_s2m = edge_index_sq2mq[0], edge_index_sq2mq[1]
  eu, em = edge_label_index[0], edge_label_index[1]

  zeros2d = jnp.zeros((_N, _HH), jnp.float32)
  ones128 = jnp.ones((_K, _HH), jnp.float32)

  mq_lo, mq_hi = _linear(mq_x, W_lin, b_lin)
  sq_lo, sq_hi = _linear(sq_x, W_lin, b_lin)

  cntb_m2s, cntb_s2m = _make_counts()(dst_m2s, dst_s2m, zeros2d, ones128)

  for l in range(4):
    agg_s_lo, agg_s_hi = _make_segsum()(
        mq_lo, mq_hi, src_m2s, dst_m2s, zeros2d)
    agg_m_lo, agg_m_hi = _make_segsum()(
        sq_lo, sq_hi, src_s2m, dst_s2m, zeros2d)
    upd = _update_relu if l == 0 else _update_plain
    sq_lo_n, sq_hi_n = upd(agg_s_lo, agg_s_hi, cntb_m2s, sq_lo, sq_hi,
                           Wl_m2s[l], bl_m2s[l], Wr_m2s[l])
    mq_lo_n, mq_hi_n = upd(agg_m_lo, agg_m_hi, cntb_s2m, mq_lo, mq_hi,
                           Wl_s2m[l], bl_s2m[l], Wr_s2m[l])
    mq_lo, mq_hi, sq_lo, sq_hi = mq_lo_n, mq_hi_n, sq_lo_n, sq_hi_n

  pout = _make_head()(mq_lo, mq_hi, sq_lo, sq_hi, eu, em)
  sel = jnp.repeat(jnp.eye(8, dtype=jnp.float32), 16, axis=0)
  return _head_reduce(pout, sel).reshape(_EL)


# speedup vs baseline: 5.0242x; 1.0516x over previous
import functools

import jax
import jax.numpy as jnp
from jax import lax
from jax.experimental import pallas as pl
from jax.experimental.pallas import tpu as pltpu
from jax.experimental.pallas import tpu_sc as plsc

_N = 10000
_E = 160000
_EL = 100000
_H = 256
_HH = 128
_DIN = 768
_K = 128
_NC = 2
_NS = 16
_ROWS_PER_TILE = 624
_ROWS_TAIL = _N - _ROWS_PER_TILE * _NS
_NCHUNK = _E // _K
_FULL_ITERS = _NCHUNK // _NS
_REM = _NCHUNK - _FULL_ITERS * _NS

@functools.lru_cache(maxsize=None)
def _mesh():
  return plsc.VectorSubcoreMesh(
      core_axis_name="c", subcore_axis_name="s",
      num_cores=_NC, num_subcores=_NS)


def _tile_rows(s, src, dst):
  r0 = s * _ROWS_PER_TILE
  pltpu.sync_copy(src.at[pl.ds(r0, _ROWS_PER_TILE)],
                  dst.at[pl.ds(r0, _ROWS_PER_TILE)])

  @pl.when(s == 0)
  def _():
    tail0 = _ROWS_PER_TILE * _NS
    pltpu.sync_copy(src.at[pl.ds(tail0, _ROWS_TAIL)],
                    dst.at[pl.ds(tail0, _ROWS_TAIL)])


def _edge_chunks(s, chunk):
  def iter_body(i, carry):
    chunk(s + i * _NS)
    return carry
  lax.fori_loop(0, _FULL_ITERS, iter_body, 0)

  @pl.when(s < _REM)
  def _():
    chunk(_FULL_ITERS * _NS + s)


@functools.lru_cache(maxsize=None)
def _make_segsum():
  base_chunks = _NCHUNK // _NS
  last_extra = _NCHUNK - base_chunks * _NS
  half = 40
  nidx = half * _K

  def body(x_lo, x_hi, src, dst, zeros2d, out_lo, out_hi,
           sbig, dbig, sv0, dv0, rv0, sv1, dv1, rv1, acc_s, g0, g1, s0, s1):
    c = lax.axis_index("c")
    s = lax.axis_index("s")

    _tile_rows(s, zeros2d, acc_s)

    ebase = s * (base_chunks * _K)
    pltpu.sync_copy(src.at[pl.ds(ebase, nidx)], sbig)
    pltpu.sync_copy(dst.at[pl.ds(ebase, nidx)], dbig)
    plsc.subcore_barrier()

    npairs_b = (base_chunks - half) // 2 + jnp.where(
        s == _NS - 1, (last_extra + 1) // 2, 0)

    def work(x_h):
      def lg(k, sv, dv, rv, gsem):
        for j in range(_K // 16):
          sl = pl.ds(j * 16, 16)
          sv[sl] = sbig[pl.ds(k * _K + j * 16, 16)]
          dv[sl] = dbig[pl.ds(k * _K + j * 16, 16)]
        pltpu.async_copy(x_h.at[sv], rv, gsem)

      def wait64k(rv, sem):
        pltpu.make_async_copy(x_h.at[pl.ds(0, _K)], rv, sem).wait()

      def sc(rv, dv, ssem):
        pltpu.async_copy(rv, acc_s.at[dv], ssem, add=True)

      def steady_pair(p, carry):
        wait64k(rv0, s0)
        lg(2 * p, sv0, dv0, rv0, g0)
        wait64k(rv1, s1)
        lg(2 * p + 1, sv1, dv1, rv1, g1)
        wait64k(rv0, g0)
        sc(rv0, dv0, s0)
        wait64k(rv1, g1)
        sc(rv1, dv1, s1)
        return carry

      lg(0, sv0, dv0, rv0, g0)
      lg(1, sv1, dv1, rv1, g1)
      wait64k(rv0, g0)
      sc(rv0, dv0, s0)
      wait64k(rv1, g1)
      sc(rv1, dv1, s1)
      lax.fori_loop(1, half // 2, steady_pair, 0)

      pltpu.sync_copy(src.at[pl.ds(ebase + nidx, nidx)], sbig)
      pltpu.sync_copy(dst.at[pl.ds(ebase + nidx, nidx)], dbig)

      lax.fori_loop(0, npairs_b, steady_pair, 0)

      wait64k(rv0, s0)
      wait64k(rv1, s1)

    @pl.when(c == 0)
    def _():
      work(x_lo)

    @pl.when(c == 1)
    def _():
      work(x_hi)

    plsc.subcore_barrier()

    @pl.when(c == 0)
    def _():
      _tile_rows(s, acc_s, out_lo)

    @pl.when(c == 1)
    def _():
      _tile_rows(s, acc_s, out_hi)

  return pl.kernel(
      body,
      out_type=[
          jax.ShapeDtypeStruct((_N, _HH), jnp.float32),
          jax.ShapeDtypeStruct((_N, _HH), jnp.float32),
      ],
      mesh=_mesh(),
      scratch_types=[
          pltpu.VMEM((nidx,), jnp.int32),
          pltpu.VMEM((nidx,), jnp.int32),
          pltpu.VMEM((_K,), jnp.int32),
          pltpu.VMEM((_K,), jnp.int32),
          pltpu.VMEM((_K, _HH), jnp.float32),
          pltpu.VMEM((_K,), jnp.int32),
          pltpu.VMEM((_K,), jnp.int32),
          pltpu.VMEM((_K, _HH), jnp.float32),
          pltpu.VMEM_SHARED((_N, _HH), jnp.float32),
          pltpu.SemaphoreType.DMA,
          pltpu.SemaphoreType.DMA,
          pltpu.SemaphoreType.DMA,
          pltpu.SemaphoreType.DMA,
      ],
      name="segsum")


@functools.lru_cache(maxsize=None)
def _make_counts():
  base_chunks = _NCHUNK // _NS
  last_extra = _NCHUNK - base_chunks * _NS
  nidx = (base_chunks + last_extra) * _K

  def body(dst_a, dst_b, zeros2d, ones128,
           out_a, out_b, dbig, dv0, dv1, rows_v, acc_s, s0, s1):
    c = lax.axis_index("c")
    s = lax.axis_index("s")

    _tile_rows(s, zeros2d, acc_s)
    pltpu.sync_copy(ones128, rows_v)
    ebase = s * (base_chunks * _K)

    @pl.when(c == 0)
    def _():
      pltpu.sync_copy(dst_a.at[pl.ds(ebase, nidx)], dbig)

    @pl.when(c == 1)
    def _():
      pltpu.sync_copy(dst_b.at[pl.ds(ebase, nidx)], dbig)

    plsc.subcore_barrier()

    npairs = (base_chunks // 2) + jnp.where(s == _NS - 1, last_extra // 2, 0)

    def stage_sc(k, dv, ssem):
      for j in range(_K // 16):
        sl = pl.ds(j * 16, 16)
        dv[sl] = dbig[pl.ds(k * _K + j * 16, 16)]
      pltpu.async_copy(rows_v, acc_s.at[dv], ssem, add=True)

    def wait64k(ssem):
      pltpu.make_async_copy(ones128, rows_v, ssem).wait()

    stage_sc(0, dv0, s0)
    stage_sc(1, dv1, s1)

    def pair(p, carry):
      wait64k(s0)
      stage_sc(2 * p, dv0, s0)
      wait64k(s1)
      stage_sc(2 * p + 1, dv1, s1)
      return carry
    lax.fori_loop(1, npairs, pair, 0)
    wait64k(s0)
    wait64k(s1)

    plsc.subcore_barrier()

    @pl.when(c == 0)
    def _():
      _tile_rows(s, acc_s, out_a)

    @pl.when(c == 1)
    def _():
      _tile_rows(s, acc_s, out_b)

  return pl.kernel(
      body,
      out_type=[
          jax.ShapeDtypeStruct((_N, _HH), jnp.float32),
          jax.ShapeDtypeStruct((_N, _HH), jnp.float32),
      ],
      mesh=_mesh(),
      scratch_types=[
          pltpu.VMEM((nidx,), jnp.int32),
          pltpu.VMEM((_K,), jnp.int32),
          pltpu.VMEM((_K,), jnp.int32),
          pltpu.VMEM((_K, _HH), jnp.float32),
          pltpu.VMEM_SHARED((_N, _HH), jnp.float32),
          pltpu.SemaphoreType.DMA,
          pltpu.SemaphoreType.DMA,
      ],
      name="edge_counts")

_HCHUNKS = _EL // _K
_HTAIL = _EL - _HCHUNKS * _K
_HITERS = -(-_HCHUNKS // (_NC * _NS))


def _head_body(mq_lo, mq_hi, sq_lo, sq_hi, eu, em, pout,
               ui, mi, ulo, uhi, mlo, mhi, res, sem):
  w = lax.axis_index("s") * _NC + lax.axis_index("c")

  def edge_partial(r):
    acc = ulo[r, pl.ds(0, 16)] * mlo[r, pl.ds(0, 16)]
    for j in range(1, _HH // 16):
      sl = pl.ds(j * 16, 16)
      acc = acc + ulo[r, sl] * mlo[r, sl]
    for j in range(_HH // 16):
      sl = pl.ds(j * 16, 16)
      acc = acc + uhi[r, sl] * mhi[r, sl]
    res[r // 8, pl.ds((r % 8) * 16, 16)] = acc

  def do_chunk(base, n):
    pltpu.sync_copy(eu.at[pl.ds(base, n)], ui.at[pl.ds(0, n)])
    pltpu.sync_copy(em.at[pl.ds(base, n)], mi.at[pl.ds(0, n)])
    pltpu.async_copy(mq_lo.at[ui], ulo, sem).wait()
    pltpu.async_copy(mq_hi.at[ui], uhi, sem).wait()
    pltpu.async_copy(sq_lo.at[mi], mlo, sem).wait()
    pltpu.async_copy(sq_hi.at[mi], mhi, sem).wait()

    def r_body(r, carry):
      edge_partial(r)
      return carry
    lax.fori_loop(0, n, r_body, 0)
    pltpu.sync_copy(res.at[pl.ds(0, n // 8)],
                    pout.at[pl.ds(pl.multiple_of(base // 8, 8), n // 8)])

  def iter_body(i, carry):
    cid = w + i * (_NC * _NS)

    @pl.when(cid < _HCHUNKS)
    def _():
      do_chunk(cid * _K, _K)
    return carry

  lax.fori_loop(0, _HITERS, iter_body, 0)

  @pl.when(w == 0)
  def _():
    do_chunk(_HCHUNKS * _K, _HTAIL)


@functools.lru_cache(maxsize=None)
def _make_head():
  return pl.kernel(
    _head_body,
    out_type=jax.ShapeDtypeStruct((_EL // 8, _HH), jnp.float32),
    mesh=_mesh(),
    scratch_types=[
        pltpu.VMEM((_K,), jnp.int32),
        pltpu.VMEM((_K,), jnp.int32),
        pltpu.VMEM((_K, _HH), jnp.float32),
        pltpu.VMEM((_K, _HH), jnp.float32),
        pltpu.VMEM((_K, _HH), jnp.float32),
        pltpu.VMEM((_K, _HH), jnp.float32),
        pltpu.VMEM((_K // 8, _HH), jnp.float32),
        pltpu.SemaphoreType.DMA,
    ],
    compiler_params=pltpu.CompilerParams(needs_layout_passes=False),
    name="edge_dot_head")


def _headsum_body(p_ref, s_ref, o_ref):
  o_ref[...] = jnp.dot(p_ref[...], s_ref[...],
                       preferred_element_type=jnp.float32)


def _head_reduce(pout, sel):
  return pl.pallas_call(
      _headsum_body,
      out_shape=jax.ShapeDtypeStruct((_EL // 8, 8), jnp.float32),
      name="head_reduce",
  )(pout, sel)


_BN = 2000


def _linear_body(x_ref, w_ref, b_ref, olo_ref, ohi_ref):
  y = (jnp.dot(x_ref[...], w_ref[...], preferred_element_type=jnp.float32)
       + b_ref[...])
  olo_ref[...] = y[:, :_HH]
  ohi_ref[...] = y[:, _HH:]


def _linear(x, W, b):
  return pl.pallas_call(
      _linear_body,
      grid=(_N // _BN,),
      in_specs=[
          pl.BlockSpec((_BN, _DIN), lambda i: (i, 0)),
          pl.BlockSpec((_DIN, _H), lambda i: (0, 0)),
          pl.BlockSpec((1, _H), lambda i: (0, 0)),
      ],
      out_specs=[
          pl.BlockSpec((_BN, _HH), lambda i: (i, 0)),
          pl.BlockSpec((_BN, _HH), lambda i: (i, 0)),
      ],
      out_shape=[
          jax.ShapeDtypeStruct((_N, _HH), jnp.float32),
          jax.ShapeDtypeStruct((_N, _HH), jnp.float32),
      ],
  )(x, W, b.reshape(1, _H))


def _make_update(relu: bool):
  def body(agg_lo_ref, agg_hi_ref, cnt_ref, x_lo_ref, x_hi_ref,
           wl_lo_ref, wl_hi_ref, wr_lo_ref, wr_hi_ref, b_ref,
           olo_ref, ohi_ref):
    inv = 1.0 / jnp.maximum(cnt_ref[...], 1.0)
    y = jnp.dot(agg_lo_ref[...] * inv, wl_lo_ref[...],
                preferred_element_type=jnp.float32)
    y += jnp.dot(agg_hi_ref[...] * inv, wl_hi_ref[...],
                 preferred_element_type=jnp.float32)
    y += jnp.dot(x_lo_ref[...], wr_lo_ref[...],
                 preferred_element_type=jnp.float32)
    y += jnp.dot(x_hi_ref[...], wr_hi_ref[...],
                 preferred_element_type=jnp.float32)
    y += b_ref[...]
    if relu:
      y = jnp.maximum(y, 0.0)
    olo_ref[...] = y[:, :_HH]
    ohi_ref[...] = y[:, _HH:]

  def update(agg_lo, agg_hi, cntb, x_lo, x_hi, Wl, bl, Wr):
    rows = lambda i: (i, 0)
    bcast = lambda i: (0, 0)
    return pl.pallas_call(
        body,
        grid=(_N // _BN,),
        in_specs=[
            pl.BlockSpec((_BN, _HH), rows),
            pl.BlockSpec((_BN, _HH), rows),
            pl.BlockSpec((_BN, _HH), rows),
            pl.BlockSpec((_BN, _HH), rows),
            pl.BlockSpec((_BN, _HH), rows),
            pl.BlockSpec((_HH, _H), bcast),
            pl.BlockSpec((_HH, _H), bcast),
            pl.BlockSpec((_HH, _H), bcast),
            pl.BlockSpec((_HH, _H), bcast),
            pl.BlockSpec((1, _H), bcast),
        ],
        out_specs=[
            pl.BlockSpec((_BN, _HH), rows),
            pl.BlockSpec((_BN, _HH), rows),
        ],
        out_shape=[
            jax.ShapeDtypeStruct((_N, _HH), jnp.float32),
            jax.ShapeDtypeStruct((_N, _HH), jnp.float32),
        ],
        name="update_relu" if relu else "update",
    )(agg_lo, agg_hi, cntb, x_lo, x_hi,
      Wl[:_HH], Wl[_HH:], Wr[:_HH], Wr[_HH:], bl.reshape(1, _H))

  return update


_update_relu = _make_update(True)
_update_plain = _make_update(False)


def kernel(mq_x, sq_x, edge_index_mq2sq, edge_index_sq2mq, edge_label_index,
           W_lin, b_lin, Wl_m2s, bl_m2s, Wr_m2s, Wl_s2m, bl_s2m, Wr_s2m):
  src_m2s, dst_m2s = edge_index_mq2sq[0], edge_index_mq2sq[1]
  src_s2m, dst_s2m = edge_index_sq2mq[0], edge_index_sq2mq[1]
  eu, em = edge_label_index[0], edge_label_index[1]

  zeros2d = jnp.zeros((_N, _HH), jnp.float32)
  ones128 = jnp.ones((_K, _HH), jnp.float32)

  mq_lo, mq_hi = _linear(mq_x, W_lin, b_lin)
  sq_lo, sq_hi = _linear(sq_x, W_lin, b_lin)

  cntb_m2s, cntb_s2m = _make_counts()(dst_m2s, dst_s2m, zeros2d, ones128)

  for l in range(4):
    agg_s_lo, agg_s_hi = _make_segsum()(
        mq_lo, mq_hi, src_m2s, dst_m2s, zeros2d)
    agg_m_lo, agg_m_hi = _make_segsum()(
        sq_lo, sq_hi, src_s2m, dst_s2m, zeros2d)
    upd = _update_relu if l == 0 else _update_plain
    sq_lo_n, sq_hi_n = upd(agg_s_lo, agg_s_hi, cntb_m2s, sq_lo, sq_hi,
                           Wl_m2s[l], bl_m2s[l], Wr_m2s[l])
    mq_lo_n, mq_hi_n = upd(agg_m_lo, agg_m_hi, cntb_s2m, mq_lo, mq_hi,
                           Wl_s2m[l], bl_s2m[l], Wr_s2m[l])
    mq_lo, mq_hi, sq_lo, sq_hi = mq_lo_n, mq_hi_n, sq_lo_n, sq_hi_n

  pout = _make_head()(mq_lo, mq_hi, sq_lo, sq_hi, eu, em)
  sel = jnp.repeat(jnp.eye(8, dtype=jnp.float32), 16, axis=0)
  return _head_reduce(pout, sel).reshape(_EL)


# speedup vs baseline: 5.3979x; 1.0744x over previous
import functools

import jax
import jax.numpy as jnp
from jax import lax
from jax.experimental import pallas as pl
from jax.experimental.pallas import tpu as pltpu
from jax.experimental.pallas import tpu_sc as plsc

_N = 10000
_E = 160000
_EL = 100000
_H = 256
_HH = 128
_DIN = 768
_K = 128
_NC = 2
_NS = 16
_ROWS_PER_TILE = 624
_ROWS_TAIL = _N - _ROWS_PER_TILE * _NS
_NCHUNK = _E // _K
_FULL_ITERS = _NCHUNK // _NS
_REM = _NCHUNK - _FULL_ITERS * _NS

@functools.lru_cache(maxsize=None)
def _mesh():
  return plsc.VectorSubcoreMesh(
      core_axis_name="c", subcore_axis_name="s",
      num_cores=_NC, num_subcores=_NS)


def _tile_rows(s, src, dst):
  r0 = s * _ROWS_PER_TILE
  pltpu.sync_copy(src.at[pl.ds(r0, _ROWS_PER_TILE)],
                  dst.at[pl.ds(r0, _ROWS_PER_TILE)])

  @pl.when(s == 0)
  def _():
    tail0 = _ROWS_PER_TILE * _NS
    pltpu.sync_copy(src.at[pl.ds(tail0, _ROWS_TAIL)],
                    dst.at[pl.ds(tail0, _ROWS_TAIL)])


def _edge_chunks(s, chunk):
  def iter_body(i, carry):
    chunk(s + i * _NS)
    return carry
  lax.fori_loop(0, _FULL_ITERS, iter_body, 0)

  @pl.when(s < _REM)
  def _():
    chunk(_FULL_ITERS * _NS + s)


@functools.lru_cache(maxsize=None)
def _make_segsum():
  base_chunks = _NCHUNK // _NS
  last_extra = _NCHUNK - base_chunks * _NS
  half = 40
  nidx = half * _K

  def body(x_lo, x_hi, src, dst, zeros2d, out_lo, out_hi,
           sbig, dbig, sv0, dv0, rv0, sv1, dv1, rv1, acc_s, g0, g1, s0, s1):
    c = lax.axis_index("c")
    s = lax.axis_index("s")

    _tile_rows(s, zeros2d, acc_s)

    ebase = s * (base_chunks * _K)
    pltpu.sync_copy(src.at[pl.ds(ebase, nidx)], sbig)
    pltpu.sync_copy(dst.at[pl.ds(ebase, nidx)], dbig)
    plsc.subcore_barrier()

    npairs_b = (base_chunks - half) // 2 + jnp.where(
        s == _NS - 1, (last_extra + 1) // 2, 0)

    def work(x_h):
      def lg(k, sv, dv, rv, gsem):
        for j in range(_K // 16):
          sl = pl.ds(j * 16, 16)
          sv[sl] = sbig[pl.ds(k * _K + j * 16, 16)]
          dv[sl] = dbig[pl.ds(k * _K + j * 16, 16)]
        pltpu.async_copy(x_h.at[sv], rv, gsem)

      def wait64k(rv, sem):
        pltpu.make_async_copy(x_h.at[pl.ds(0, _K)], rv, sem).wait()

      def sc(rv, dv, ssem):
        pltpu.async_copy(rv, acc_s.at[dv], ssem, add=True)

      def steady_pair(p, carry):
        wait64k(rv0, s0)
        lg(2 * p, sv0, dv0, rv0, g0)
        wait64k(rv1, s1)
        lg(2 * p + 1, sv1, dv1, rv1, g1)
        wait64k(rv0, g0)
        sc(rv0, dv0, s0)
        wait64k(rv1, g1)
        sc(rv1, dv1, s1)
        return carry

      lg(0, sv0, dv0, rv0, g0)
      lg(1, sv1, dv1, rv1, g1)
      wait64k(rv0, g0)
      sc(rv0, dv0, s0)
      wait64k(rv1, g1)
      sc(rv1, dv1, s1)
      lax.fori_loop(1, half // 2, steady_pair, 0)

      pltpu.sync_copy(src.at[pl.ds(ebase + nidx, nidx)], sbig)
      pltpu.sync_copy(dst.at[pl.ds(ebase + nidx, nidx)], dbig)

      lax.fori_loop(0, npairs_b, steady_pair, 0)

      wait64k(rv0, s0)
      wait64k(rv1, s1)

    @pl.when(c == 0)
    def _():
      work(x_lo)

    @pl.when(c == 1)
    def _():
      work(x_hi)

    plsc.subcore_barrier()

    @pl.when(c == 0)
    def _():
      _tile_rows(s, acc_s, out_lo)

    @pl.when(c == 1)
    def _():
      _tile_rows(s, acc_s, out_hi)

  return pl.kernel(
      body,
      out_type=[
          jax.ShapeDtypeStruct((_N, _HH), jnp.float32),
          jax.ShapeDtypeStruct((_N, _HH), jnp.float32),
      ],
      mesh=_mesh(),
      scratch_types=[
          pltpu.VMEM((nidx,), jnp.int32),
          pltpu.VMEM((nidx,), jnp.int32),
          pltpu.VMEM((_K,), jnp.int32),
          pltpu.VMEM((_K,), jnp.int32),
          pltpu.VMEM((_K, _HH), jnp.float32),
          pltpu.VMEM((_K,), jnp.int32),
          pltpu.VMEM((_K,), jnp.int32),
          pltpu.VMEM((_K, _HH), jnp.float32),
          pltpu.VMEM_SHARED((_N, _HH), jnp.float32),
          pltpu.SemaphoreType.DMA,
          pltpu.SemaphoreType.DMA,
          pltpu.SemaphoreType.DMA,
          pltpu.SemaphoreType.DMA,
      ],
      name="segsum")


@functools.lru_cache(maxsize=None)
def _make_counts():
  base_chunks = _NCHUNK // _NS
  last_extra = _NCHUNK - base_chunks * _NS
  nidx = (base_chunks + last_extra) * _K

  def body(dst_a, dst_b, zeros2d, ones128,
           out_a, out_b, dbig, dv0, dv1, rows_v, acc_s, s0, s1):
    c = lax.axis_index("c")
    s = lax.axis_index("s")

    _tile_rows(s, zeros2d, acc_s)
    pltpu.sync_copy(ones128, rows_v)
    ebase = s * (base_chunks * _K)

    @pl.when(c == 0)
    def _():
      pltpu.sync_copy(dst_a.at[pl.ds(ebase, nidx)], dbig)

    @pl.when(c == 1)
    def _():
      pltpu.sync_copy(dst_b.at[pl.ds(ebase, nidx)], dbig)

    plsc.subcore_barrier()

    npairs = (base_chunks // 2) + jnp.where(s == _NS - 1, last_extra // 2, 0)

    def stage_sc(k, dv, ssem):
      for j in range(_K // 16):
        sl = pl.ds(j * 16, 16)
        dv[sl] = dbig[pl.ds(k * _K + j * 16, 16)]
      pltpu.async_copy(rows_v, acc_s.at[dv], ssem, add=True)

    def wait64k(ssem):
      pltpu.make_async_copy(ones128, rows_v, ssem).wait()

    stage_sc(0, dv0, s0)
    stage_sc(1, dv1, s1)

    def pair(p, carry):
      wait64k(s0)
      stage_sc(2 * p, dv0, s0)
      wait64k(s1)
      stage_sc(2 * p + 1, dv1, s1)
      return carry
    lax.fori_loop(1, npairs, pair, 0)
    wait64k(s0)
    wait64k(s1)

    plsc.subcore_barrier()

    @pl.when(c == 0)
    def _():
      _tile_rows(s, acc_s, out_a)

    @pl.when(c == 1)
    def _():
      _tile_rows(s, acc_s, out_b)

  return pl.kernel(
      body,
      out_type=[
          jax.ShapeDtypeStruct((_N, _HH), jnp.float32),
          jax.ShapeDtypeStruct((_N, _HH), jnp.float32),
      ],
      mesh=_mesh(),
      scratch_types=[
          pltpu.VMEM((nidx,), jnp.int32),
          pltpu.VMEM((_K,), jnp.int32),
          pltpu.VMEM((_K,), jnp.int32),
          pltpu.VMEM((_K, _HH), jnp.float32),
          pltpu.VMEM_SHARED((_N, _HH), jnp.float32),
          pltpu.SemaphoreType.DMA,
          pltpu.SemaphoreType.DMA,
      ],
      name="edge_counts")

_HK = 64
_HCHUNKS = _EL // _HK
_HTAIL = _EL - _HCHUNKS * _HK
_HITERS = -(-_HCHUNKS // (_NC * _NS))


def _head_body(mq_lo, mq_hi, sq_lo, sq_hi, eu, em, pout, bufs_a, bufs_b,
               g_a, g_b, o_a, o_b):
  w = lax.axis_index("s") * _NC + lax.axis_index("c")

  def start(cid, bufs, gsem):
    ui, mi, ulo, uhi, mlo, mhi, res = bufs
    base = cid * _HK
    pltpu.sync_copy(eu.at[pl.ds(base, _HK)], ui)
    pltpu.sync_copy(em.at[pl.ds(base, _HK)], mi)
    pltpu.async_copy(mq_lo.at[ui], ulo, gsem)
    pltpu.async_copy(mq_hi.at[ui], uhi, gsem)
    pltpu.async_copy(sq_lo.at[mi], mlo, gsem)
    pltpu.async_copy(sq_hi.at[mi], mhi, gsem)

  def wait_gathers(bufs, gsem):
    _, _, ulo, uhi, mlo, mhi, _ = bufs
    pltpu.make_async_copy(mq_lo.at[pl.ds(0, _HK)], ulo, gsem).wait()
    pltpu.make_async_copy(mq_hi.at[pl.ds(0, _HK)], uhi, gsem).wait()
    pltpu.make_async_copy(sq_lo.at[pl.ds(0, _HK)], mlo, gsem).wait()
    pltpu.make_async_copy(sq_hi.at[pl.ds(0, _HK)], mhi, gsem).wait()

  def compute_store(cid, n, bufs, osem):
    _, _, ulo, uhi, mlo, mhi, res = bufs

    def edge_partial(r, carry):
      acc = ulo[r, pl.ds(0, 16)] * mlo[r, pl.ds(0, 16)]
      for j in range(1, _HH // 16):
        sl = pl.ds(j * 16, 16)
        acc = acc + ulo[r, sl] * mlo[r, sl]
      for j in range(_HH // 16):
        sl = pl.ds(j * 16, 16)
        acc = acc + uhi[r, sl] * mhi[r, sl]
      res[r // 8, pl.ds((r % 8) * 16, 16)] = acc
      return carry

    lax.fori_loop(0, n, edge_partial, 0)
    pltpu.async_copy(
        res.at[pl.ds(0, n // 8)],
        pout.at[pl.ds(pl.multiple_of(cid * (_HK // 8), 8), n // 8)], osem)

  def wait_out(bufs, osem, n=_HK):
    pltpu.make_async_copy(mq_lo.at[pl.ds(0, n // 8)],
                          bufs[6].at[pl.ds(0, n // 8)], osem).wait()

  nw = _NC * _NS
  start(w, bufs_a, g_a)
  start(w + nw, bufs_b, g_b)
  wait_gathers(bufs_a, g_a)
  compute_store(w, _HK, bufs_a, o_a)

  def step(i, bufs, gsem, osem, bufs_nxt, g_nxt, o_nxt):
    cid = w + i * nw

    @pl.when(cid < _HCHUNKS)
    def _():
      nxt = cid + nw

      @pl.when(nxt < _HCHUNKS)
      def _():
        wait_out(bufs_nxt, o_nxt)
        start(nxt, bufs_nxt, g_nxt)
      wait_gathers(bufs, gsem)
      compute_store(cid, _HK, bufs, osem)

  def pair_body(p, carry):
    step(2 * p + 1, bufs_b, g_b, o_b, bufs_a, g_a, o_a)
    step(2 * p + 2, bufs_a, g_a, o_a, bufs_b, g_b, o_b)
    return carry

  lax.fori_loop(0, _HITERS // 2, pair_body, 0)

  wait_out(bufs_a, o_a)
  wait_out(bufs_b, o_b)

  @pl.when(w == 0)
  def _():
    ui, mi = bufs_a[0], bufs_a[1]
    base = _HCHUNKS * _HK
    pltpu.sync_copy(eu.at[pl.ds(base, _HTAIL)], ui.at[pl.ds(0, _HTAIL)])
    pltpu.sync_copy(em.at[pl.ds(base, _HTAIL)], mi.at[pl.ds(0, _HTAIL)])
    start_tail = pltpu.async_copy
    start_tail(mq_lo.at[ui], bufs_a[2], g_a)
    start_tail(mq_hi.at[ui], bufs_a[3], g_a)
    start_tail(sq_lo.at[mi], bufs_a[4], g_a)
    start_tail(sq_hi.at[mi], bufs_a[5], g_a)
    wait_gathers(bufs_a, g_a)
    compute_store(_HCHUNKS, _HTAIL, bufs_a, o_a)
    wait_out(bufs_a, o_a, _HTAIL)


def _head_bufset():
  return [
      pltpu.VMEM((_HK,), jnp.int32),
      pltpu.VMEM((_HK,), jnp.int32),
      pltpu.VMEM((_HK, _HH), jnp.float32),
      pltpu.VMEM((_HK, _HH), jnp.float32),
      pltpu.VMEM((_HK, _HH), jnp.float32),
      pltpu.VMEM((_HK, _HH), jnp.float32),
      pltpu.VMEM((_HK // 8, _HH), jnp.float32),
  ]


@functools.lru_cache(maxsize=None)
def _make_head():
  return pl.kernel(
    _head_body,
    out_type=jax.ShapeDtypeStruct((_EL // 8, _HH), jnp.float32),
    mesh=_mesh(),
    scratch_types=[
        _head_bufset(),
        _head_bufset(),
        pltpu.SemaphoreType.DMA,
        pltpu.SemaphoreType.DMA,
        pltpu.SemaphoreType.DMA,
        pltpu.SemaphoreType.DMA,
    ],
    compiler_params=pltpu.CompilerParams(needs_layout_passes=False),
    name="edge_dot_head")


def _headsum_body(p_ref, s_ref, o_ref):
  o_ref[...] = jnp.dot(p_ref[...], s_ref[...],
                       preferred_element_type=jnp.float32)


def _head_reduce(pout, sel):
  return pl.pallas_call(
      _headsum_body,
      out_shape=jax.ShapeDtypeStruct((_EL // 8, 8), jnp.float32),
      name="head_reduce",
  )(pout, sel)


_BN = 2000


def _linear_body(x_ref, w_ref, b_ref, olo_ref, ohi_ref):
  y = (jnp.dot(x_ref[...], w_ref[...], preferred_element_type=jnp.float32)
       + b_ref[...])
  olo_ref[...] = y[:, :_HH]
  ohi_ref[...] = y[:, _HH:]


def _linear(x, W, b):
  return pl.pallas_call(
      _linear_body,
      grid=(_N // _BN,),
      in_specs=[
          pl.BlockSpec((_BN, _DIN), lambda i: (i, 0)),
          pl.BlockSpec((_DIN, _H), lambda i: (0, 0)),
          pl.BlockSpec((1, _H), lambda i: (0, 0)),
      ],
      out_specs=[
          pl.BlockSpec((_BN, _HH), lambda i: (i, 0)),
          pl.BlockSpec((_BN, _HH), lambda i: (i, 0)),
      ],
      out_shape=[
          jax.ShapeDtypeStruct((_N, _HH), jnp.float32),
          jax.ShapeDtypeStruct((_N, _HH), jnp.float32),
      ],
  )(x, W, b.reshape(1, _H))


def _make_update(relu: bool):
  def body(agg_lo_ref, agg_hi_ref, cnt_ref, x_lo_ref, x_hi_ref,
           wl_lo_ref, wl_hi_ref, wr_lo_ref, wr_hi_ref, b_ref,
           olo_ref, ohi_ref):
    inv = 1.0 / jnp.maximum(cnt_ref[...], 1.0)
    y = jnp.dot(agg_lo_ref[...] * inv, wl_lo_ref[...],
                preferred_element_type=jnp.float32)
    y += jnp.dot(agg_hi_ref[...] * inv, wl_hi_ref[...],
                 preferred_element_type=jnp.float32)
    y += jnp.dot(x_lo_ref[...], wr_lo_ref[...],
                 preferred_element_type=jnp.float32)
    y += jnp.dot(x_hi_ref[...], wr_hi_ref[...],
                 preferred_element_type=jnp.float32)
    y += b_ref[...]
    if relu:
      y = jnp.maximum(y, 0.0)
    olo_ref[...] = y[:, :_HH]
    ohi_ref[...] = y[:, _HH:]

  def update(agg_lo, agg_hi, cntb, x_lo, x_hi, Wl, bl, Wr):
    rows = lambda i: (i, 0)
    bcast = lambda i: (0, 0)
    return pl.pallas_call(
        body,
        grid=(_N // _BN,),
        in_specs=[
            pl.BlockSpec((_BN, _HH), rows),
            pl.BlockSpec((_BN, _HH), rows),
            pl.BlockSpec((_BN, _HH), rows),
            pl.BlockSpec((_BN, _HH), rows),
            pl.BlockSpec((_BN, _HH), rows),
            pl.BlockSpec((_HH, _H), bcast),
            pl.BlockSpec((_HH, _H), bcast),
            pl.BlockSpec((_HH, _H), bcast),
            pl.BlockSpec((_HH, _H), bcast),
            pl.BlockSpec((1, _H), bcast),
        ],
        out_specs=[
            pl.BlockSpec((_BN, _HH), rows),
            pl.BlockSpec((_BN, _HH), rows),
        ],
        out_shape=[
            jax.ShapeDtypeStruct((_N, _HH), jnp.float32),
            jax.ShapeDtypeStruct((_N, _HH), jnp.float32),
        ],
        name="update_relu" if relu else "update",
    )(agg_lo, agg_hi, cntb, x_lo, x_hi,
      Wl[:_HH], Wl[_HH:], Wr[:_HH], Wr[_HH:], bl.reshape(1, _H))

  return update


_update_relu = _make_update(True)
_update_plain = _make_update(False)


def kernel(mq_x, sq_x, edge_index_mq2sq, edge_index_sq2mq, edge_label_index,
           W_lin, b_lin, Wl_m2s, bl_m2s, Wr_m2s, Wl_s2m, bl_s2m, Wr_s2m):
  src_m2s, dst_m2s = edge_index_mq2sq[0], edge_index_mq2sq[1]
  src_s2m, dst_s2m = edge_index_sq2mq[0], edge_index_sq2mq[1]
  eu, em = edge_label_index[0], edge_label_index[1]

  zeros2d = jnp.zeros((_N, _HH), jnp.float32)
  ones128 = jnp.ones((_K, _HH), jnp.float32)

  mq_lo, mq_hi = _linear(mq_x, W_lin, b_lin)
  sq_lo, sq_hi = _linear(sq_x, W_lin, b_lin)

  cntb_m2s, cntb_s2m = _make_counts()(dst_m2s, dst_s2m, zeros2d, ones128)

  for l in range(4):
    agg_s_lo, agg_s_hi = _make_segsum()(
        mq_lo, mq_hi, src_m2s, dst_m2s, zeros2d)
    agg_m_lo, agg_m_hi = _make_segsum()(
        sq_lo, sq_hi, src_s2m, dst_s2m, zeros2d)
    upd = _update_relu if l == 0 else _update_plain
    sq_lo_n, sq_hi_n = upd(agg_s_lo, agg_s_hi, cntb_m2s, sq_lo, sq_hi,
                           Wl_m2s[l], bl_m2s[l], Wr_m2s[l])
    mq_lo_n, mq_hi_n = upd(agg_m_lo, agg_m_hi, cntb_s2m, mq_lo, mq_hi,
                           Wl_s2m[l], bl_s2m[l], Wr_s2m[l])
    mq_lo, mq_hi, sq_lo, sq_hi = mq_lo_n, mq_hi_n, sq_lo_n, sq_hi_n

  pout = _make_head()(mq_lo, mq_hi, sq_lo, sq_hi, eu, em)
  sel = jnp.repeat(jnp.eye(8, dtype=jnp.float32), 16, axis=0)
  return _head_reduce(pout, sel).reshape(_EL)


# speedup vs baseline: 5.4038x; 1.0011x over previous
import functools

import jax
import jax.numpy as jnp
from jax import lax
from jax.experimental import pallas as pl
from jax.experimental.pallas import tpu as pltpu
from jax.experimental.pallas import tpu_sc as plsc

_N = 10000
_E = 160000
_EL = 100000
_H = 256
_HH = 128
_DIN = 768
_K = 128
_NC = 2
_NS = 16
_ROWS_PER_TILE = 624
_ROWS_TAIL = _N - _ROWS_PER_TILE * _NS
_NCHUNK = _E // _K
_FULL_ITERS = _NCHUNK // _NS
_REM = _NCHUNK - _FULL_ITERS * _NS

@functools.lru_cache(maxsize=None)
def _mesh():
  return plsc.VectorSubcoreMesh(
      core_axis_name="c", subcore_axis_name="s",
      num_cores=_NC, num_subcores=_NS)


def _tile_rows(s, src, dst):
  r0 = s * _ROWS_PER_TILE
  pltpu.sync_copy(src.at[pl.ds(r0, _ROWS_PER_TILE)],
                  dst.at[pl.ds(r0, _ROWS_PER_TILE)])

  @pl.when(s == 0)
  def _():
    tail0 = _ROWS_PER_TILE * _NS
    pltpu.sync_copy(src.at[pl.ds(tail0, _ROWS_TAIL)],
                    dst.at[pl.ds(tail0, _ROWS_TAIL)])


def _edge_chunks(s, chunk):
  def iter_body(i, carry):
    chunk(s + i * _NS)
    return carry
  lax.fori_loop(0, _FULL_ITERS, iter_body, 0)

  @pl.when(s < _REM)
  def _():
    chunk(_FULL_ITERS * _NS + s)


@functools.lru_cache(maxsize=None)
def _make_segsum():
  base_chunks = _NCHUNK // _NS
  last_extra = _NCHUNK - base_chunks * _NS
  half = 40
  nidx = half * _K

  def body(x_lo, x_hi, src, dst, zeros2d, out_lo, out_hi,
           sbig, dbig, sv0a, sv0b, dv0, rv0, sv1a, sv1b, dv1, rv1,
           acc_s, g0, g1, s0, s1):
    c = lax.axis_index("c")
    s = lax.axis_index("s")

    _tile_rows(s, zeros2d, acc_s)

    ebase = s * (base_chunks * _K)
    pltpu.sync_copy(src.at[pl.ds(ebase, nidx)], sbig)
    pltpu.sync_copy(dst.at[pl.ds(ebase, nidx)], dbig)
    plsc.subcore_barrier()

    npairs_b = (base_chunks - half) // 2 + jnp.where(
        s == _NS - 1, (last_extra + 1) // 2, 0)

    def work(x_h):
      def lg(k, sva, svb, dv, rv, gsem):
        for j in range(_K // 32):
          sl = pl.ds(j * 16, 16)
          sva[sl] = sbig[pl.ds(k * _K + j * 16, 16)]
          svb[sl] = sbig[pl.ds(k * _K + 64 + j * 16, 16)]
        for j in range(_K // 16):
          dv[pl.ds(j * 16, 16)] = dbig[pl.ds(k * _K + j * 16, 16)]
        pltpu.async_copy(x_h.at[sva], rv.at[pl.ds(0, 64)], gsem)
        pltpu.async_copy(x_h.at[svb], rv.at[pl.ds(64, 64)], gsem)

      def wait64k(rv, sem):
        pltpu.make_async_copy(x_h.at[pl.ds(0, _K)], rv, sem).wait()

      def sc(rv, dv, ssem):
        pltpu.async_copy(rv, acc_s.at[dv], ssem, add=True)

      def steady_pair(p, carry):
        wait64k(rv0, s0)
        lg(2 * p, sv0a, sv0b, dv0, rv0, g0)
        wait64k(rv1, s1)
        lg(2 * p + 1, sv1a, sv1b, dv1, rv1, g1)
        wait64k(rv0, g0)
        sc(rv0, dv0, s0)
        wait64k(rv1, g1)
        sc(rv1, dv1, s1)
        return carry

      lg(0, sv0a, sv0b, dv0, rv0, g0)
      lg(1, sv1a, sv1b, dv1, rv1, g1)
      wait64k(rv0, g0)
      sc(rv0, dv0, s0)
      wait64k(rv1, g1)
      sc(rv1, dv1, s1)
      lax.fori_loop(1, half // 2, steady_pair, 0)

      pltpu.sync_copy(src.at[pl.ds(ebase + nidx, nidx)], sbig)
      pltpu.sync_copy(dst.at[pl.ds(ebase + nidx, nidx)], dbig)

      lax.fori_loop(0, npairs_b, steady_pair, 0)

      wait64k(rv0, s0)
      wait64k(rv1, s1)

    @pl.when(c == 0)
    def _():
      work(x_lo)

    @pl.when(c == 1)
    def _():
      work(x_hi)

    plsc.subcore_barrier()

    @pl.when(c == 0)
    def _():
      _tile_rows(s, acc_s, out_lo)

    @pl.when(c == 1)
    def _():
      _tile_rows(s, acc_s, out_hi)

  return pl.kernel(
      body,
      out_type=[
          jax.ShapeDtypeStruct((_N, _HH), jnp.float32),
          jax.ShapeDtypeStruct((_N, _HH), jnp.float32),
      ],
      mesh=_mesh(),
      scratch_types=[
          pltpu.VMEM((nidx,), jnp.int32),
          pltpu.VMEM((nidx,), jnp.int32),
          pltpu.VMEM((_K // 2,), jnp.int32),
          pltpu.VMEM((_K // 2,), jnp.int32),
          pltpu.VMEM((_K,), jnp.int32),
          pltpu.VMEM((_K, _HH), jnp.float32),
          pltpu.VMEM((_K // 2,), jnp.int32),
          pltpu.VMEM((_K // 2,), jnp.int32),
          pltpu.VMEM((_K,), jnp.int32),
          pltpu.VMEM((_K, _HH), jnp.float32),
          pltpu.VMEM_SHARED((_N, _HH), jnp.float32),
          pltpu.SemaphoreType.DMA,
          pltpu.SemaphoreType.DMA,
          pltpu.SemaphoreType.DMA,
          pltpu.SemaphoreType.DMA,
      ],
      name="segsum")


@functools.lru_cache(maxsize=None)
def _make_counts():
  base_chunks = _NCHUNK // _NS
  last_extra = _NCHUNK - base_chunks * _NS
  nidx = (base_chunks + last_extra) * _K

  def body(dst_a, dst_b, zeros2d, ones128,
           out_a, out_b, dbig, dv0, dv1, rows_v, acc_s, s0, s1):
    c = lax.axis_index("c")
    s = lax.axis_index("s")

    _tile_rows(s, zeros2d, acc_s)
    pltpu.sync_copy(ones128, rows_v)
    ebase = s * (base_chunks * _K)

    @pl.when(c == 0)
    def _():
      pltpu.sync_copy(dst_a.at[pl.ds(ebase, nidx)], dbig)

    @pl.when(c == 1)
    def _():
      pltpu.sync_copy(dst_b.at[pl.ds(ebase, nidx)], dbig)

    plsc.subcore_barrier()

    npairs = (base_chunks // 2) + jnp.where(s == _NS - 1, last_extra // 2, 0)

    def stage_sc(k, dv, ssem):
      for j in range(_K // 16):
        sl = pl.ds(j * 16, 16)
        dv[sl] = dbig[pl.ds(k * _K + j * 16, 16)]
      pltpu.async_copy(rows_v, acc_s.at[dv], ssem, add=True)

    def wait64k(ssem):
      pltpu.make_async_copy(ones128, rows_v, ssem).wait()

    stage_sc(0, dv0, s0)
    stage_sc(1, dv1, s1)

    def pair(p, carry):
      wait64k(s0)
      stage_sc(2 * p, dv0, s0)
      wait64k(s1)
      stage_sc(2 * p + 1, dv1, s1)
      return carry
    lax.fori_loop(1, npairs, pair, 0)
    wait64k(s0)
    wait64k(s1)

    plsc.subcore_barrier()

    @pl.when(c == 0)
    def _():
      _tile_rows(s, acc_s, out_a)

    @pl.when(c == 1)
    def _():
      _tile_rows(s, acc_s, out_b)

  return pl.kernel(
      body,
      out_type=[
          jax.ShapeDtypeStruct((_N, _HH), jnp.float32),
          jax.ShapeDtypeStruct((_N, _HH), jnp.float32),
      ],
      mesh=_mesh(),
      scratch_types=[
          pltpu.VMEM((nidx,), jnp.int32),
          pltpu.VMEM((_K,), jnp.int32),
          pltpu.VMEM((_K,), jnp.int32),
          pltpu.VMEM((_K, _HH), jnp.float32),
          pltpu.VMEM_SHARED((_N, _HH), jnp.float32),
          pltpu.SemaphoreType.DMA,
          pltpu.SemaphoreType.DMA,
      ],
      name="edge_counts")

_HK = 64
_HCHUNKS = _EL // _HK
_HTAIL = _EL - _HCHUNKS * _HK
_HITERS = -(-_HCHUNKS // (_NC * _NS))


def _head_body(mq_lo, mq_hi, sq_lo, sq_hi, eu, em, pout, bufs_a, bufs_b,
               g_a, g_b, o_a, o_b):
  w = lax.axis_index("s") * _NC + lax.axis_index("c")

  def start(cid, bufs, gsem):
    ui, mi, ulo, uhi, mlo, mhi, res = bufs
    base = cid * _HK
    pltpu.sync_copy(eu.at[pl.ds(base, _HK)], ui)
    pltpu.sync_copy(em.at[pl.ds(base, _HK)], mi)
    pltpu.async_copy(mq_lo.at[ui], ulo, gsem)
    pltpu.async_copy(mq_hi.at[ui], uhi, gsem)
    pltpu.async_copy(sq_lo.at[mi], mlo, gsem)
    pltpu.async_copy(sq_hi.at[mi], mhi, gsem)

  def wait_gathers(bufs, gsem):
    _, _, ulo, uhi, mlo, mhi, _ = bufs
    pltpu.make_async_copy(mq_lo.at[pl.ds(0, _HK)], ulo, gsem).wait()
    pltpu.make_async_copy(mq_hi.at[pl.ds(0, _HK)], uhi, gsem).wait()
    pltpu.make_async_copy(sq_lo.at[pl.ds(0, _HK)], mlo, gsem).wait()
    pltpu.make_async_copy(sq_hi.at[pl.ds(0, _HK)], mhi, gsem).wait()

  def compute_store(cid, n, bufs, osem):
    _, _, ulo, uhi, mlo, mhi, res = bufs

    def edge_partial(r, carry):
      acc = ulo[r, pl.ds(0, 16)] * mlo[r, pl.ds(0, 16)]
      for j in range(1, _HH // 16):
        sl = pl.ds(j * 16, 16)
        acc = acc + ulo[r, sl] * mlo[r, sl]
      for j in range(_HH // 16):
        sl = pl.ds(j * 16, 16)
        acc = acc + uhi[r, sl] * mhi[r, sl]
      res[r // 8, pl.ds((r % 8) * 16, 16)] = acc
      return carry

    lax.fori_loop(0, n, edge_partial, 0)
    pltpu.async_copy(
        res.at[pl.ds(0, n // 8)],
        pout.at[pl.ds(pl.multiple_of(cid * (_HK // 8), 8), n // 8)], osem)

  def wait_out(bufs, osem, n=_HK):
    pltpu.make_async_copy(mq_lo.at[pl.ds(0, n // 8)],
                          bufs[6].at[pl.ds(0, n // 8)], osem).wait()

  nw = _NC * _NS
  start(w, bufs_a, g_a)
  start(w + nw, bufs_b, g_b)
  wait_gathers(bufs_a, g_a)
  compute_store(w, _HK, bufs_a, o_a)

  def step(i, bufs, gsem, osem, bufs_nxt, g_nxt, o_nxt):
    cid = w + i * nw

    @pl.when(cid < _HCHUNKS)
    def _():
      nxt = cid + nw

      @pl.when(nxt < _HCHUNKS)
      def _():
        wait_out(bufs_nxt, o_nxt)
        start(nxt, bufs_nxt, g_nxt)
      wait_gathers(bufs, gsem)
      compute_store(cid, _HK, bufs, osem)

  def pair_body(p, carry):
    step(2 * p + 1, bufs_b, g_b, o_b, bufs_a, g_a, o_a)
    step(2 * p + 2, bufs_a, g_a, o_a, bufs_b, g_b, o_b)
    return carry

  lax.fori_loop(0, _HITERS // 2, pair_body, 0)

  wait_out(bufs_a, o_a)
  wait_out(bufs_b, o_b)

  @pl.when(w == 0)
  def _():
    ui, mi = bufs_a[0], bufs_a[1]
    base = _HCHUNKS * _HK
    pltpu.sync_copy(eu.at[pl.ds(base, _HTAIL)], ui.at[pl.ds(0, _HTAIL)])
    pltpu.sync_copy(em.at[pl.ds(base, _HTAIL)], mi.at[pl.ds(0, _HTAIL)])
    start_tail = pltpu.async_copy
    start_tail(mq_lo.at[ui], bufs_a[2], g_a)
    start_tail(mq_hi.at[ui], bufs_a[3], g_a)
    start_tail(sq_lo.at[mi], bufs_a[4], g_a)
    start_tail(sq_hi.at[mi], bufs_a[5], g_a)
    wait_gathers(bufs_a, g_a)
    compute_store(_HCHUNKS, _HTAIL, bufs_a, o_a)
    wait_out(bufs_a, o_a, _HTAIL)


def _head_bufset():
  return [
      pltpu.VMEM((_HK,), jnp.int32),
      pltpu.VMEM((_HK,), jnp.int32),
      pltpu.VMEM((_HK, _HH), jnp.float32),
      pltpu.VMEM((_HK, _HH), jnp.float32),
      pltpu.VMEM((_HK, _HH), jnp.float32),
      pltpu.VMEM((_HK, _HH), jnp.float32),
      pltpu.VMEM((_HK // 8, _HH), jnp.float32),
  ]


@functools.lru_cache(maxsize=None)
def _make_head():
  return pl.kernel(
    _head_body,
    out_type=jax.ShapeDtypeStruct((_EL // 8, _HH), jnp.float32),
    mesh=_mesh(),
    scratch_types=[
        _head_bufset(),
        _head_bufset(),
        pltpu.SemaphoreType.DMA,
        pltpu.SemaphoreType.DMA,
        pltpu.SemaphoreType.DMA,
        pltpu.SemaphoreType.DMA,
    ],
    compiler_params=pltpu.CompilerParams(needs_layout_passes=False),
    name="edge_dot_head")


def _headsum_body(p_ref, s_ref, o_ref):
  o_ref[...] = jnp.dot(p_ref[...], s_ref[...],
                       preferred_element_type=jnp.float32)


def _head_reduce(pout, sel):
  return pl.pallas_call(
      _headsum_body,
      out_shape=jax.ShapeDtypeStruct((_EL // 8, 8), jnp.float32),
      name="head_reduce",
  )(pout, sel)


_BN = 2000


def _linear_body(x_ref, w_ref, b_ref, olo_ref, ohi_ref):
  y = (jnp.dot(x_ref[...], w_ref[...], preferred_element_type=jnp.float32)
       + b_ref[...])
  olo_ref[...] = y[:, :_HH]
  ohi_ref[...] = y[:, _HH:]


def _linear(x, W, b):
  return pl.pallas_call(
      _linear_body,
      grid=(_N // _BN,),
      in_specs=[
          pl.BlockSpec((_BN, _DIN), lambda i: (i, 0)),
          pl.BlockSpec((_DIN, _H), lambda i: (0, 0)),
          pl.BlockSpec((1, _H), lambda i: (0, 0)),
      ],
      out_specs=[
          pl.BlockSpec((_BN, _HH), lambda i: (i, 0)),
          pl.BlockSpec((_BN, _HH), lambda i: (i, 0)),
      ],
      out_shape=[
          jax.ShapeDtypeStruct((_N, _HH), jnp.float32),
          jax.ShapeDtypeStruct((_N, _HH), jnp.float32),
      ],
  )(x, W, b.reshape(1, _H))


def _make_update(relu: bool):
  def body(agg_lo_ref, agg_hi_ref, cnt_ref, x_lo_ref, x_hi_ref,
           wl_lo_ref, wl_hi_ref, wr_lo_ref, wr_hi_ref, b_ref,
           olo_ref, ohi_ref):
    inv = 1.0 / jnp.maximum(cnt_ref[...], 1.0)
    y = jnp.dot(agg_lo_ref[...] * inv, wl_lo_ref[...],
                preferred_element_type=jnp.float32)
    y += jnp.dot(agg_hi_ref[...] * inv, wl_hi_ref[...],
                 preferred_element_type=jnp.float32)
    y += jnp.dot(x_lo_ref[...], wr_lo_ref[...],
                 preferred_element_type=jnp.float32)
    y += jnp.dot(x_hi_ref[...], wr_hi_ref[...],
                 preferred_element_type=jnp.float32)
    y += b_ref[...]
    if relu:
      y = jnp.maximum(y, 0.0)
    olo_ref[...] = y[:, :_HH]
    ohi_ref[...] = y[:, _HH:]

  def update(agg_lo, agg_hi, cntb, x_lo, x_hi, Wl, bl, Wr):
    rows = lambda i: (i, 0)
    bcast = lambda i: (0, 0)
    return pl.pallas_call(
        body,
        grid=(_N // _BN,),
        in_specs=[
            pl.BlockSpec((_BN, _HH), rows),
            pl.BlockSpec((_BN, _HH), rows),
            pl.BlockSpec((_BN, _HH), rows),
            pl.BlockSpec((_BN, _HH), rows),
            pl.BlockSpec((_BN, _HH), rows),
            pl.BlockSpec((_HH, _H), bcast),
            pl.BlockSpec((_HH, _H), bcast),
            pl.BlockSpec((_HH, _H), bcast),
            pl.BlockSpec((_HH, _H), bcast),
            pl.BlockSpec((1, _H), bcast),
        ],
        out_specs=[
            pl.BlockSpec((_BN, _HH), rows),
            pl.BlockSpec((_BN, _HH), rows),
        ],
        out_shape=[
            jax.ShapeDtypeStruct((_N, _HH), jnp.float32),
            jax.ShapeDtypeStruct((_N, _HH), jnp.float32),
        ],
        name="update_relu" if relu else "update",
    )(agg_lo, agg_hi, cntb, x_lo, x_hi,
      Wl[:_HH], Wl[_HH:], Wr[:_HH], Wr[_HH:], bl.reshape(1, _H))

  return update


_update_relu = _make_update(True)
_update_plain = _make_update(False)


def kernel(mq_x, sq_x, edge_index_mq2sq, edge_index_sq2mq, edge_label_index,
           W_lin, b_lin, Wl_m2s, bl_m2s, Wr_m2s, Wl_s2m, bl_s2m, Wr_s2m):
  src_m2s, dst_m2s = edge_index_mq2sq[0], edge_index_mq2sq[1]
  src_s2m, dst_s2m = edge_index_sq2mq[0], edge_index_sq2mq[1]
  eu, em = edge_label_index[0], edge_label_index[1]

  zeros2d = jnp.zeros((_N, _HH), jnp.float32)
  ones128 = jnp.ones((_K, _HH), jnp.float32)

  mq_lo, mq_hi = _linear(mq_x, W_lin, b_lin)
  sq_lo, sq_hi = _linear(sq_x, W_lin, b_lin)

  cntb_m2s, cntb_s2m = _make_counts()(dst_m2s, dst_s2m, zeros2d, ones128)

  for l in range(4):
    agg_s_lo, agg_s_hi = _make_segsum()(
        mq_lo, mq_hi, src_m2s, dst_m2s, zeros2d)
    agg_m_lo, agg_m_hi = _make_segsum()(
        sq_lo, sq_hi, src_s2m, dst_s2m, zeros2d)
    upd = _update_relu if l == 0 else _update_plain
    sq_lo_n, sq_hi_n = upd(agg_s_lo, agg_s_hi, cntb_m2s, sq_lo, sq_hi,
                           Wl_m2s[l], bl_m2s[l], Wr_m2s[l])
    mq_lo_n, mq_hi_n = upd(agg_m_lo, agg_m_hi, cntb_s2m, mq_lo, mq_hi,
                           Wl_s2m[l], bl_s2m[l], Wr_s2m[l])
    mq_lo, mq_hi, sq_lo, sq_hi = mq_lo_n, mq_hi_n, sq_lo_n, sq_hi_n

  pout = _make_head()(mq_lo, mq_hi, sq_lo, sq_hi, eu, em)
  sel = jnp.repeat(jnp.eye(8, dtype=jnp.float32), 16, axis=0)
  return _head_reduce(pout, sel).reshape(_EL)
